# Initial kernel scaffold; baseline (speedup 1.0000x reference)
import jax
import jax.numpy as jnp
from jax.experimental import pallas as pl


def kernel(edge_index, h, e, Wh, We, Wa, att, gamma_h, beta_h, gamma_e, beta_e):
    raise NotImplementedError("write your pallas kernel here")


# speedup vs baseline: 4.0458x; 4.0458x over previous
import functools

import jax
import jax.numpy as jnp
from jax import lax
from jax.experimental import pallas as pl
from jax.experimental.pallas import tpu as pltpu
from jax.experimental.pallas import tpu_sc as plsc

N = 10000
E = 320000
D = 128
L = 2
NPAD = 10240

NC = 2
NS = 16
NW = NC * NS
EW = E // NW
CH = 80
NCHW = EW // CH
NROWS = E // CH

_PREC = lax.Precision.HIGHEST

_mesh = plsc.VectorSubcoreMesh(core_axis_name="c", subcore_axis_name="s")
_SC_PARAMS = pltpu.CompilerParams(use_tc_tiling_on_sc=False)
_SC_PARAMS_NL = pltpu.CompilerParams(use_tc_tiling_on_sc=False,
                                     needs_layout_passes=False)


def _f32(shape):
    return jax.ShapeDtypeStruct(shape, jnp.float32)


def _node_mm_body(h_ref, wa_ref, wh_ref, hwa_ref, hwh2_ref):
    h = h_ref[...]
    hwa_ref[...] = lax.dot_general(h, wa_ref[...], (((1,), (0,)), ((), ())),
                                   precision=_PREC,
                                   preferred_element_type=jnp.float32)
    hwh = lax.dot_general(h, wh_ref[...], (((1,), (0,)), ((), ())),
                          precision=_PREC,
                          preferred_element_type=jnp.float32)
    hwh2_ref[0] = hwh[:, :64]
    hwh2_ref[1] = hwh[:, 64:]


def _node_mm(h, wa, wh):
    bn = 2000
    return pl.pallas_call(
        _node_mm_body,
        grid=(N // bn,),
        in_specs=[
            pl.BlockSpec((bn, D), lambda i: (i, 0)),
            pl.BlockSpec((D, D), lambda i: (0, 0)),
            pl.BlockSpec((D, D), lambda i: (0, 0)),
        ],
        out_specs=[
            pl.BlockSpec((bn, D), lambda i: (i, 0)),
            pl.BlockSpec((NC, bn, 64), lambda i: (0, i, 0)),
        ],
        out_shape=[_f32((N, D)), _f32((NC, N, 64))],
    )(h, wa, wh)


@functools.partial(
    pl.kernel,
    out_type=[_f32((E, D)), _f32((E, D))],
    mesh=_mesh,
    compiler_params=_SC_PARAMS,
    scratch_types=[
        pltpu.VMEM((CH,), jnp.int32),
        pltpu.VMEM((CH,), jnp.int32),
        pltpu.VMEM((CH, D), jnp.float32),
        pltpu.VMEM((CH, D), jnp.float32),
        pltpu.SemaphoreType.DMA,
        pltpu.SemaphoreType.DMA,
    ],
)
def _sc_gather(hwa_hbm, src_hbm, dst_hbm, gs_hbm, gd_hbm, ia, ib, ra, rb,
               sa, sb):
    wid = lax.axis_index("s") * NC + lax.axis_index("c")

    @pl.loop(0, NCHW)
    def _(k):
        row = wid * NCHW + k
        b = row * CH
        pltpu.sync_copy(src_hbm.at[row], ia)
        pltpu.sync_copy(dst_hbm.at[row], ib)
        c1 = pltpu.async_copy(hwa_hbm.at[ia], ra, sa)
        c2 = pltpu.async_copy(hwa_hbm.at[ib], rb, sb)
        c1.wait()
        c2.wait()
        pltpu.sync_copy(ra, gs_hbm.at[pl.ds(b, CH)])
        pltpu.sync_copy(rb, gd_hbm.at[pl.ds(b, CH)])


_BE = 1600
_NBE = E // _BE


def _edge_dense_body(e_ref, gs_ref, gd_ref, we_ref, att_ref,
                     en_ref, s_ref, st_ref, acc_ref):
    i = pl.program_id(0)
    en = lax.dot_general(e_ref[...], we_ref[...], (((1,), (0,)), ((), ())),
                         precision=_PREC, preferred_element_type=jnp.float32)
    en = en + gs_ref[...] + gd_ref[...]
    en_ref[...] = en
    lr = jnp.where(en > 0, en, 0.2 * en)
    s = jnp.sum(lr * att_ref[0, :][None, :], axis=1)
    s_ref[0, 0, :] = s

    @pl.when(i == 0)
    def _():
        acc_ref[...] = jnp.zeros_like(acc_ref)

    colsum = jnp.sum(en, axis=0)
    colsq = jnp.sum(en * en, axis=0)
    acc_ref[0, :] += colsum
    acc_ref[1, :] += colsq

    @pl.when(i == _NBE - 1)
    def _():
        st_ref[...] = acc_ref[...]


def _edge_dense(e, gs, gd, we, att):
    return pl.pallas_call(
        _edge_dense_body,
        grid=(_NBE,),
        in_specs=[
            pl.BlockSpec((_BE, D), lambda i: (i, 0)),
            pl.BlockSpec((_BE, D), lambda i: (i, 0)),
            pl.BlockSpec((_BE, D), lambda i: (i, 0)),
            pl.BlockSpec((D, D), lambda i: (0, 0)),
            pl.BlockSpec((1, D), lambda i: (0, 0)),
        ],
        out_specs=[
            pl.BlockSpec((_BE, D), lambda i: (i, 0)),
            pl.BlockSpec((1, 1, _BE), lambda i: (i, 0, 0)),
            pl.BlockSpec((8, D), lambda i: (0, 0)),
        ],
        out_shape=[_f32((E, D)), _f32((_NBE, 1, _BE)), _f32((8, D))],
        scratch_shapes=[pltpu.VMEM((8, D), jnp.float32)],
    )(e, gs, gd, we, att)


_RB = 25
_NRB = NCHW // _RB
_NSL = NPAD // NS


@functools.partial(
    pl.kernel,
    out_type=[_f32((NROWS, CH)), _f32((NC, NPAD))],
    mesh=_mesh,
    compiler_params=_SC_PARAMS,
    scratch_types=[
        pltpu.VMEM((_RB, CH), jnp.float32),
        pltpu.VMEM((_RB, CH), jnp.int32),
        pltpu.VMEM((_NSL,), jnp.float32),
        pltpu.VMEM_SHARED((NPAD,), jnp.float32),
        pltpu.SemaphoreType.DMA,
    ],
)
def _sc_denom(s_hbm, dst_hbm, ex_hbm, pden_hbm, sv, dv, zb, den_sh, sem):
    c = lax.axis_index("c")
    sid = lax.axis_index("s")
    wid = sid * NC + c

    @pl.loop(0, _NSL, step=16)
    def _(j):
        zb[pl.ds(j, 16)] = jnp.zeros((16,), jnp.float32)

    pltpu.sync_copy(zb, den_sh.at[pl.ds(sid * _NSL, _NSL)])
    plsc.subcore_barrier()

    @pl.loop(0, _NRB)
    def _(r):
        row0 = wid * NCHW + r * _RB
        pltpu.sync_copy(s_hbm.at[pl.ds(row0, _RB)], sv)
        pltpu.sync_copy(dst_hbm.at[pl.ds(row0, _RB)], dv)

        @pl.loop(0, _RB)
        def _(j):
            for k in range(CH // 16):
                sl = pl.ds(k * 16, 16)
                sv[j, sl] = jnp.exp(sv[j, sl])

        pltpu.sync_copy(sv, ex_hbm.at[pl.ds(row0, _RB)])

        @pl.loop(0, _RB)
        def _(j):
            pltpu.sync_copy(sv.at[j], den_sh.at[dv.at[j]], add=True)

    plsc.subcore_barrier()
    pltpu.sync_copy(den_sh.at[pl.ds(sid * _NSL, _NSL)], zb)
    pltpu.sync_copy(zb, pden_hbm.at[c, pl.ds(sid * _NSL, _NSL)])


_RC = 5
_RT = NROWS // NS
_NRC = _RT // _RC
_DH = D // NC


@functools.partial(
    pl.kernel,
    out_type=_f32((NC, NPAD, _DH)),
    mesh=_mesh,
    compiler_params=_SC_PARAMS_NL,
    scratch_types=[
        pltpu.VMEM((_RC, CH), jnp.int32),
        pltpu.VMEM((_RC, CH), jnp.int32),
        pltpu.VMEM((_RC, CH), jnp.float32),
        pltpu.VMEM((_RC * CH, _DH), jnp.float32),
        pltpu.VMEM((NPAD,), jnp.float32),
        pltpu.VMEM((NPAD,), jnp.float32),
        pltpu.VMEM_SHARED((NPAD, _DH), jnp.float32),
        pltpu.SemaphoreType.DMA,
    ],
)
def _sc_message(hwh_hbm, src_hbm, dst_hbm, ex_hbm, pden_hbm, hpart_hbm,
                srcv, dstv, exv, rows, den, den2, hacc_sh, sem):
    c = lax.axis_index("c")
    sid = lax.axis_index("s")

    @pl.loop(0, _RC * CH)
    def _(j):
        for k in range(_DH // 16):
            rows[j, pl.ds(k * 16, 16)] = jnp.zeros((16,), jnp.float32)

    pltpu.sync_copy(rows.at[pl.ds(0, 400)],
                    hacc_sh.at[pl.ds(sid * _NSL, 400)])
    pltpu.sync_copy(rows.at[pl.ds(0, 240)],
                    hacc_sh.at[pl.ds(sid * _NSL + 400, 240)])

    pltpu.sync_copy(pden_hbm.at[0], den)
    pltpu.sync_copy(pden_hbm.at[1], den2)

    @pl.loop(0, NPAD, step=16)
    def _(j):
        sl = pl.ds(j, 16)
        den[sl] = den[sl] + den2[sl]

    plsc.subcore_barrier()

    coff = c * N

    @pl.loop(0, _NRC)
    def _(r):
        row0 = sid * _RT + r * _RC
        pltpu.sync_copy(src_hbm.at[pl.ds(row0, _RC)], srcv)
        pltpu.sync_copy(dst_hbm.at[pl.ds(row0, _RC)], dstv)
        pltpu.sync_copy(ex_hbm.at[pl.ds(row0, _RC)], exv)

        @pl.loop(0, _RC)
        def _(j):
            for k in range(CH // 16):
                sl = pl.ds(k * 16, 16)
                srcv[j, sl] = srcv[j, sl] + coff

        cps = [
            pltpu.async_copy(hwh_hbm.at[srcv.at[j]],
                             rows.at[pl.ds(j * CH, CH)], sem)
            for j in range(_RC)
        ]

        @pl.loop(0, _RC)
        def _(j):
            for k in range(CH // 16):
                sl = pl.ds(k * 16, 16)
                d16 = plsc.load_gather(den, [dstv[j, sl]])
                exv[j, sl] = exv[j, sl] / d16

        for cp in cps:
            cp.wait()

        @pl.loop(0, _RC)
        def _(j):
            @pl.loop(0, CH)
            def _(t):
                a16 = plsc.load_gather(exv.at[j],
                                       [jnp.full((16,), t, jnp.int32)])
                for k in range(_DH // 16):
                    sl = pl.ds(k * 16, 16)
                    rows[j * CH + t, sl] = rows[j * CH + t, sl] * a16

        for j in range(_RC):
            pltpu.sync_copy(rows.at[pl.ds(j * CH, CH)],
                            hacc_sh.at[dstv.at[j]], add=True)

    plsc.subcore_barrier()

    pltpu.sync_copy(hacc_sh.at[pl.ds(sid * _NSL, 400)],
                    rows.at[pl.ds(0, 400)])
    pltpu.sync_copy(rows.at[pl.ds(0, 400)],
                    hpart_hbm.at[c, pl.ds(sid * _NSL, 400)])
    pltpu.sync_copy(hacc_sh.at[pl.ds(sid * _NSL + 400, 240)],
                    rows.at[pl.ds(0, 240)])
    pltpu.sync_copy(rows.at[pl.ds(0, 240)],
                    hpart_hbm.at[c, pl.ds(sid * _NSL + 400, 240)])


def _bn_apply_body(n_rows, x_ref, xn_ref, st_ref, g_ref, b_ref, out_ref):
    mu = st_ref[0, :] / n_rows
    var = st_ref[1, :] / n_rows - mu * mu
    inv = lax.rsqrt(var + 1e-5)
    scale = g_ref[0, :] * inv
    shift = b_ref[0, :] - mu * scale
    y = xn_ref[...] * scale[None, :] + shift[None, :]
    out_ref[...] = x_ref[...] + jnp.maximum(y, 0.0)


def _bn_apply(x, xn, st, g, b, n_rows, bm):
    m = x.shape[0]
    return pl.pallas_call(
        functools.partial(_bn_apply_body, float(n_rows)),
        grid=(m // bm,),
        in_specs=[
            pl.BlockSpec((bm, D), lambda i: (i, 0)),
            pl.BlockSpec((bm, D), lambda i: (i, 0)),
            pl.BlockSpec((8, D), lambda i: (0, 0)),
            pl.BlockSpec((1, D), lambda i: (0, 0)),
            pl.BlockSpec((1, D), lambda i: (0, 0)),
        ],
        out_specs=pl.BlockSpec((bm, D), lambda i: (i, 0)),
        out_shape=_f32((m, D)),
    )(x, xn, st, g, b)


_BH = 1280
_NBH = NPAD // _BH


def _h_sum_body(hp_ref, hn_ref, st_ref, acc_ref):
    i = pl.program_id(0)
    hn = jnp.concatenate([hp_ref[0], hp_ref[1]], axis=1)
    hn_ref[...] = hn

    @pl.when(i == 0)
    def _():
        acc_ref[...] = jnp.zeros_like(acc_ref)

    acc_ref[0, :] += jnp.sum(hn, axis=0)
    acc_ref[1, :] += jnp.sum(hn * hn, axis=0)

    @pl.when(i == _NBH - 1)
    def _():
        st_ref[...] = acc_ref[...]


def _h_sum(hpart):
    return pl.pallas_call(
        _h_sum_body,
        grid=(_NBH,),
        in_specs=[pl.BlockSpec((NC, _BH, _DH), lambda i: (0, i, 0))],
        out_specs=[
            pl.BlockSpec((_BH, D), lambda i: (i, 0)),
            pl.BlockSpec((8, D), lambda i: (0, 0)),
        ],
        out_shape=[_f32((NPAD, D)), _f32((8, D))],
        scratch_shapes=[pltpu.VMEM((8, D), jnp.float32)],
    )(hpart)


def _layer(src, dst, src2, dst2, h, e, wh, we, wa, att, gh, bh, ge, be):
    hwa, hwh = _node_mm(h, wa, wh)
    gs, gd = _sc_gather(hwa, src2, dst2)
    en, s3, st_e = _edge_dense(e, gs, gd, we, att.reshape(1, D))
    s2 = s3.reshape(NROWS, CH)
    ex, pden = _sc_denom(s2, dst2)
    hpart = _sc_message(hwh.reshape(NC * N, _DH), src2, dst2, ex, pden)
    e_out = _bn_apply(e, en, st_e, ge.reshape(1, D), be.reshape(1, D), E, _BE)
    hn, st_h = _h_sum(hpart)
    h_out = _bn_apply(h, hn[:N], st_h, gh.reshape(1, D), bh.reshape(1, D),
                      N, 1000)
    return h_out, e_out


def kernel(edge_index, h, e, Wh, We, Wa, att, gamma_h, beta_h, gamma_e,
           beta_e):
    src = edge_index[0].astype(jnp.int32)
    dst = edge_index[1].astype(jnp.int32)
    src2 = src.reshape(NROWS, CH)
    dst2 = dst.reshape(NROWS, CH)
    for i in range(L):
        h, e = _layer(src, dst, src2, dst2, h, e, Wh[i], We[i], Wa[i],
                      att[i], gamma_h[i], beta_h[i], gamma_e[i], beta_e[i])
    return (h, e)


# speedup vs baseline: 5.6685x; 1.4011x over previous
import functools

import jax
import jax.numpy as jnp
from jax import lax
from jax.experimental import pallas as pl
from jax.experimental.pallas import tpu as pltpu
from jax.experimental.pallas import tpu_sc as plsc

N = 10000
E = 320000
D = 128
L = 2
NPAD = 10240

NC = 2
NS = 16
NW = NC * NS
EW = E // NW
CH = 80
NCHW = EW // CH
NROWS = E // CH
_NSL = NPAD // NS

_PREC = lax.Precision.HIGHEST

_mesh = plsc.VectorSubcoreMesh(core_axis_name="c", subcore_axis_name="s")
_SC_PARAMS = pltpu.CompilerParams(use_tc_tiling_on_sc=False)
_SC_PARAMS_NL = pltpu.CompilerParams(use_tc_tiling_on_sc=False,
                                     needs_layout_passes=False)


def _f32(shape):
    return jax.ShapeDtypeStruct(shape, jnp.float32)


def _node_mm_body(h_ref, wa_ref, wh_ref, hwa_ref, hwh2_ref):
    h = h_ref[...]
    hwa_ref[...] = lax.dot_general(h, wa_ref[...], (((1,), (0,)), ((), ())),
                                   precision=_PREC,
                                   preferred_element_type=jnp.float32)
    hwh = lax.dot_general(h, wh_ref[...], (((1,), (0,)), ((), ())),
                          precision=_PREC,
                          preferred_element_type=jnp.float32)
    hwh2_ref[0] = hwh[:, :64]
    hwh2_ref[1] = hwh[:, 64:]


def _node_mm(h, wa, wh):
    bn = 2000
    return pl.pallas_call(
        _node_mm_body,
        grid=(N // bn,),
        in_specs=[
            pl.BlockSpec((bn, D), lambda i: (i, 0)),
            pl.BlockSpec((D, D), lambda i: (0, 0)),
            pl.BlockSpec((D, D), lambda i: (0, 0)),
        ],
        out_specs=[
            pl.BlockSpec((bn, D), lambda i: (i, 0)),
            pl.BlockSpec((NC, bn, 64), lambda i: (0, i, 0)),
        ],
        out_shape=[_f32((N, D)), _f32((NC, N, 64))],
    )(h, wa, wh)


_GA = 5
_NRA = NCHW // _GA


@functools.partial(
    pl.kernel,
    out_type=[_f32((E, D)), _f32((E, D))],
    mesh=_mesh,
    compiler_params=_SC_PARAMS,
    scratch_types=[
        pltpu.VMEM((_GA, CH), jnp.int32),
        pltpu.VMEM((_GA, CH), jnp.int32),
        pltpu.VMEM((_GA * CH, D), jnp.float32),
        pltpu.VMEM((_GA * CH, D), jnp.float32),
        pltpu.SemaphoreType.DMA,
        pltpu.SemaphoreType.DMA,
    ],
)
def _sc_gather(hwa_hbm, src_hbm, dst_hbm, gs_hbm, gd_hbm, ia, ib, ra, rb,
               sg, sw):
    wid = lax.axis_index("s") * NC + lax.axis_index("c")
    row0 = wid * NCHW

    @pl.loop(0, _NRA)
    def _(r):
        rr = row0 + r * _GA
        pltpu.sync_copy(src_hbm.at[pl.ds(rr, _GA)], ia)
        pltpu.sync_copy(dst_hbm.at[pl.ds(rr, _GA)], ib)
        cps = []
        for j in range(_GA):
            cps.append(pltpu.async_copy(hwa_hbm.at[ia.at[j]],
                                        ra.at[pl.ds(j * CH, CH)], sg))
            cps.append(pltpu.async_copy(hwa_hbm.at[ib.at[j]],
                                        rb.at[pl.ds(j * CH, CH)], sg))
        for cp in cps:
            cp.wait()
        b = rr * CH
        w1 = pltpu.async_copy(ra, gs_hbm.at[pl.ds(b, _GA * CH)], sw)
        w2 = pltpu.async_copy(rb, gd_hbm.at[pl.ds(b, _GA * CH)], sw)
        w1.wait()
        w2.wait()


_BE = 1600
_NBE = E // _BE


def _edge_dense_body(e_ref, gs_ref, gd_ref, we_ref, att_ref,
                     en_ref, s_ref, st_ref, acc_ref):
    i = pl.program_id(0)
    en = lax.dot_general(e_ref[...], we_ref[...], (((1,), (0,)), ((), ())),
                         precision=lax.Precision.DEFAULT,
                         preferred_element_type=jnp.float32)
    en = en + gs_ref[...] + gd_ref[...]
    en_ref[...] = en
    lr = jnp.where(en > 0, en, 0.2 * en)
    s = jnp.sum(lr * att_ref[0, :][None, :], axis=1)
    s_ref[0, 0, :] = s

    @pl.when(i == 0)
    def _():
        acc_ref[...] = jnp.zeros_like(acc_ref)

    acc_ref[0, :] += jnp.sum(en, axis=0)
    acc_ref[1, :] += jnp.sum(en * en, axis=0)

    @pl.when(i == _NBE - 1)
    def _():
        st_ref[...] = acc_ref[...]


def _edge_dense(e, gs, gd, we, att):
    return pl.pallas_call(
        _edge_dense_body,
        grid=(_NBE,),
        in_specs=[
            pl.BlockSpec((_BE, D), lambda i: (i, 0)),
            pl.BlockSpec((_BE, D), lambda i: (i, 0)),
            pl.BlockSpec((_BE, D), lambda i: (i, 0)),
            pl.BlockSpec((D, D), lambda i: (0, 0)),
            pl.BlockSpec((1, D), lambda i: (0, 0)),
        ],
        out_specs=[
            pl.BlockSpec((_BE, D), lambda i: (i, 0)),
            pl.BlockSpec((1, 1, _BE), lambda i: (i, 0, 0)),
            pl.BlockSpec((8, D), lambda i: (0, 0)),
        ],
        out_shape=[_f32((E, D)), _f32((_NBE, 1, _BE)), _f32((8, D))],
        scratch_shapes=[pltpu.VMEM((8, D), jnp.float32)],
    )(e, gs, gd, we, att)


_RT = NROWS // NS
_GC = 5
_NR2 = _RT // _GC
_DH = D // NC


@functools.partial(
    pl.kernel,
    out_type=[_f32((NC, NPAD, _DH)), _f32((NPAD,))],
    mesh=_mesh,
    compiler_params=_SC_PARAMS_NL,
    scratch_types=[
        pltpu.VMEM((_GC, CH), jnp.int32),
        pltpu.VMEM((_GC, CH), jnp.int32),
        pltpu.VMEM((_GC, CH), jnp.float32),
        pltpu.VMEM((_GC, CH), jnp.int32),
        pltpu.VMEM((_GC, CH), jnp.int32),
        pltpu.VMEM((_GC, CH), jnp.float32),
        pltpu.VMEM((10, CH), jnp.float32),
        pltpu.VMEM((10, CH), jnp.int32),
        pltpu.VMEM((_GC * CH, _DH), jnp.float32),
        pltpu.VMEM((_GC * CH, _DH), jnp.float32),
        pltpu.VMEM((_NSL,), jnp.float32),
        pltpu.VMEM_SHARED((NPAD,), jnp.float32),
        pltpu.VMEM_SHARED((NPAD, _DH), jnp.float32),
        pltpu.SemaphoreType.DMA,
        pltpu.SemaphoreType.DMA,
        pltpu.SemaphoreType.DMA,
    ],
)
def _sc_message(hwh_hbm, s_hbm, src_hbm, dst_hbm, hpart_hbm, den_hbm,
                bi0, bd0, be0, bi1, bd1, be1, sv, dv, r0, r1, zb,
                den_sh, hacc_sh, sg0, sg1, ss):
    c = lax.axis_index("c")
    sid = lax.axis_index("s")
    row0 = sid * _RT
    coff = c * N

    @pl.loop(0, _NSL, step=16)
    def _(j):
        zb[pl.ds(j, 16)] = jnp.zeros((16,), jnp.float32)

    pltpu.sync_copy(zb, den_sh.at[pl.ds(sid * _NSL, _NSL)])

    @pl.loop(0, _GC * CH)
    def _(j):
        for k in range(_DH // 16):
            r0[j, pl.ds(k * 16, 16)] = jnp.zeros((16,), jnp.float32)

    pltpu.sync_copy(r0.at[pl.ds(0, 400)],
                    hacc_sh.at[pl.ds(sid * _NSL, 400)])
    pltpu.sync_copy(r0.at[pl.ds(0, 240)],
                    hacc_sh.at[pl.ds(sid * _NSL + 400, 240)])

    plsc.subcore_barrier()

    @pl.loop(0, _RT // 10)
    def _(g):
        rr = row0 + g * 10
        pltpu.sync_copy(s_hbm.at[pl.ds(rr, 10)], sv)
        pltpu.sync_copy(dst_hbm.at[pl.ds(rr, 10)], dv)

        @pl.loop(0, 10)
        def _(j):
            for k in range(CH // 16):
                sl = pl.ds(k * 16, 16)
                sv[j, sl] = jnp.exp(sv[j, sl])

        cps = [
            pltpu.async_copy(sv.at[j], den_sh.at[dv.at[j]], ss, add=True)
            for j in range(10)
        ]
        for cp in cps:
            cp.wait()

    plsc.subcore_barrier()

    @pl.when(c == 0)
    def _():
        pltpu.sync_copy(den_sh.at[pl.ds(sid * _NSL, _NSL)], zb)
        pltpu.sync_copy(zb, den_hbm.at[pl.ds(sid * _NSL, _NSL)])

    def _issue(r, bi, bd, be, buf, sem):
        rr = row0 + r * _GC
        pltpu.sync_copy(src_hbm.at[pl.ds(rr, _GC)], bi)
        pltpu.sync_copy(dst_hbm.at[pl.ds(rr, _GC)], bd)
        pltpu.sync_copy(s_hbm.at[pl.ds(rr, _GC)], be)

        @pl.loop(0, _GC)
        def _(j):
            for k in range(CH // 16):
                sl = pl.ds(k * 16, 16)
                bi[j, sl] = bi[j, sl] + coff
                be[j, sl] = jnp.exp(be[j, sl])

        for j in range(_GC):
            pltpu.async_copy(hwh_hbm.at[bi.at[j]],
                             buf.at[pl.ds(j * CH, CH)], sem)

    def _drain(buf, sem):
        pltpu.make_async_copy(hwh_hbm.at[pl.ds(0, _GC * CH)], buf, sem).wait()

    def _scale_scatter(bd, be, buf):
        for j in range(_GC):
            @pl.loop(0, CH)
            def _(t):
                a16 = plsc.load_gather(be.at[j],
                                       [jnp.full((16,), t, jnp.int32)])
                for k in range(_DH // 16):
                    sl = pl.ds(k * 16, 16)
                    buf[j * CH + t, sl] = buf[j * CH + t, sl] * a16

        for j in range(_GC):
            pltpu.sync_copy(buf.at[pl.ds(j * CH, CH)],
                            hacc_sh.at[bd.at[j]], add=True)

    _issue(0, bi0, bd0, be0, r0, sg0)

    @pl.loop(0, _NR2 // 2)
    def _(i):
        ra = 2 * i
        rb = 2 * i + 1
        _issue(rb, bi1, bd1, be1, r1, sg1)
        _drain(r0, sg0)
        _scale_scatter(bd0, be0, r0)

        @pl.when(ra + 2 < _NR2)
        def _():
            _issue(ra + 2, bi0, bd0, be0, r0, sg0)

        _drain(r1, sg1)
        _scale_scatter(bd1, be1, r1)

    plsc.subcore_barrier()

    pltpu.sync_copy(hacc_sh.at[pl.ds(sid * _NSL, 400)],
                    r0.at[pl.ds(0, 400)])
    pltpu.sync_copy(r0.at[pl.ds(0, 400)],
                    hpart_hbm.at[c, pl.ds(sid * _NSL, 400)])
    pltpu.sync_copy(hacc_sh.at[pl.ds(sid * _NSL + 400, 240)],
                    r0.at[pl.ds(0, 240)])
    pltpu.sync_copy(r0.at[pl.ds(0, 240)],
                    hpart_hbm.at[c, pl.ds(sid * _NSL + 400, 240)])


def _bn_apply_body(n_rows, x_ref, xn_ref, st_ref, g_ref, b_ref, out_ref):
    mu = st_ref[0, :] / n_rows
    var = st_ref[1, :] / n_rows - mu * mu
    inv = lax.rsqrt(var + 1e-5)
    scale = g_ref[0, :] * inv
    shift = b_ref[0, :] - mu * scale
    y = xn_ref[...] * scale[None, :] + shift[None, :]
    out_ref[...] = x_ref[...] + jnp.maximum(y, 0.0)


def _bn_apply(x, xn, st, g, b, n_rows, bm):
    m = x.shape[0]
    return pl.pallas_call(
        functools.partial(_bn_apply_body, float(n_rows)),
        grid=(m // bm,),
        in_specs=[
            pl.BlockSpec((bm, D), lambda i: (i, 0)),
            pl.BlockSpec((bm, D), lambda i: (i, 0)),
            pl.BlockSpec((8, D), lambda i: (0, 0)),
            pl.BlockSpec((1, D), lambda i: (0, 0)),
            pl.BlockSpec((1, D), lambda i: (0, 0)),
        ],
        out_specs=pl.BlockSpec((bm, D), lambda i: (i, 0)),
        out_shape=_f32((m, D)),
    )(x, xn, st, g, b)


_BH = 1280
_NBH = NPAD // _BH


def _h_sum_body(hp_ref, den_ref, hn_ref, st_ref, acc_ref):
    i = pl.program_id(0)
    hn = jnp.concatenate([hp_ref[0], hp_ref[1]], axis=1)
    den = den_ref[0, 0, :]
    rden = jnp.where(den > 0, 1.0 / den, 0.0)
    hn = hn * rden[:, None]
    hn_ref[...] = hn

    @pl.when(i == 0)
    def _():
        acc_ref[...] = jnp.zeros_like(acc_ref)

    acc_ref[0, :] += jnp.sum(hn, axis=0)
    acc_ref[1, :] += jnp.sum(hn * hn, axis=0)

    @pl.when(i == _NBH - 1)
    def _():
        st_ref[...] = acc_ref[...]


def _h_sum(hpart, den):
    return pl.pallas_call(
        _h_sum_body,
        grid=(_NBH,),
        in_specs=[
            pl.BlockSpec((NC, _BH, _DH), lambda i: (0, i, 0)),
            pl.BlockSpec((1, 1, _BH), lambda i: (i, 0, 0)),
        ],
        out_specs=[
            pl.BlockSpec((_BH, D), lambda i: (i, 0)),
            pl.BlockSpec((8, D), lambda i: (0, 0)),
        ],
        out_shape=[_f32((NPAD, D)), _f32((8, D))],
        scratch_shapes=[pltpu.VMEM((8, D), jnp.float32)],
    )(hpart, den)


def _layer(src2, dst2, h, e, wh, we, wa, att, gh, bh, ge, be):
    hwa, hwh = _node_mm(h, wa, wh)
    gs, gd = _sc_gather(hwa, src2, dst2)
    en, s3, st_e = _edge_dense(e, gs, gd, we, att.reshape(1, D))
    s2 = s3.reshape(NROWS, CH)
    hpart, den = _sc_message(hwh.reshape(NC * N, _DH), s2, src2, dst2)
    e_out = _bn_apply(e, en, st_e, ge.reshape(1, D), be.reshape(1, D), E, _BE)
    hn, st_h = _h_sum(hpart, den.reshape(_NBH, 1, _BH))
    h_out = _bn_apply(h, hn[:N], st_h, gh.reshape(1, D), bh.reshape(1, D),
                      N, 1000)
    return h_out, e_out


def kernel(edge_index, h, e, Wh, We, Wa, att, gamma_h, beta_h, gamma_e,
           beta_e):
    src2 = edge_index[0].astype(jnp.int32).reshape(NROWS, CH)
    dst2 = edge_index[1].astype(jnp.int32).reshape(NROWS, CH)
    for i in range(L):
        h, e = _layer(src2, dst2, h, e, Wh[i], We[i], Wa[i],
                      att[i], gamma_h[i], beta_h[i], gamma_e[i], beta_e[i])
    return (h, e)


# speedup vs baseline: 5.7883x; 1.0211x over previous
import functools

import jax
import jax.numpy as jnp
from jax import lax
from jax.experimental import pallas as pl
from jax.experimental.pallas import tpu as pltpu
from jax.experimental.pallas import tpu_sc as plsc

N = 10000
E = 320000
D = 128
L = 2
NPAD = 10240

NC = 2
NS = 16
NW = NC * NS
EW = E // NW
CH = 80
NCHW = EW // CH
NROWS = E // CH
_NSL = NPAD // NS

_PREC = lax.Precision.HIGHEST

_mesh = plsc.VectorSubcoreMesh(core_axis_name="c", subcore_axis_name="s")
_SC_PARAMS = pltpu.CompilerParams(use_tc_tiling_on_sc=False)
_SC_PARAMS_NL = pltpu.CompilerParams(use_tc_tiling_on_sc=False,
                                     needs_layout_passes=False)


def _f32(shape):
    return jax.ShapeDtypeStruct(shape, jnp.float32)


def _node_mm_body(h_ref, wa_ref, wh_ref, hwa_ref, hwh2_ref):
    h = h_ref[...]
    hwa_ref[...] = lax.dot_general(h, wa_ref[...], (((1,), (0,)), ((), ())),
                                   precision=_PREC,
                                   preferred_element_type=jnp.float32)
    hwh = lax.dot_general(h, wh_ref[...], (((1,), (0,)), ((), ())),
                          precision=_PREC,
                          preferred_element_type=jnp.float32)
    hwh2_ref[0] = hwh[:, :64]
    hwh2_ref[1] = hwh[:, 64:]


def _node_mm(h, wa, wh):
    bn = 2000
    return pl.pallas_call(
        _node_mm_body,
        grid=(N // bn,),
        in_specs=[
            pl.BlockSpec((bn, D), lambda i: (i, 0)),
            pl.BlockSpec((D, D), lambda i: (0, 0)),
            pl.BlockSpec((D, D), lambda i: (0, 0)),
        ],
        out_specs=[
            pl.BlockSpec((bn, D), lambda i: (i, 0)),
            pl.BlockSpec((NC, bn, 64), lambda i: (0, i, 0)),
        ],
        out_shape=[_f32((N, D)), _f32((NC, N, 64))],
    )(h, wa, wh)


_GA = 2
_NRA = 62


@functools.partial(
    pl.kernel,
    out_type=[_f32((E, D)), _f32((E, D))],
    mesh=_mesh,
    compiler_params=_SC_PARAMS,
    scratch_types=[
        pltpu.VMEM((_GA, CH), jnp.int32),
        pltpu.VMEM((_GA, CH), jnp.int32),
        pltpu.VMEM((_GA, CH), jnp.int32),
        pltpu.VMEM((_GA, CH), jnp.int32),
        pltpu.VMEM((_GA * CH, D), jnp.float32),
        pltpu.VMEM((_GA * CH, D), jnp.float32),
        pltpu.VMEM((_GA * CH, D), jnp.float32),
        pltpu.VMEM((_GA * CH, D), jnp.float32),
        pltpu.SemaphoreType.DMA,
        pltpu.SemaphoreType.DMA,
        pltpu.SemaphoreType.DMA,
        pltpu.SemaphoreType.DMA,
    ],
)
def _sc_gather(hwa_hbm, src_hbm, dst_hbm, gs_hbm, gd_hbm,
               ia0, ib0, ia1, ib1, ras0, rad0, ras1, rad1,
               sg0, sg1, sw0, sw1):
    wid = lax.axis_index("s") * NC + lax.axis_index("c")
    row0 = wid * NCHW
    nb = _GA * CH

    def _issue(r, ia, ib, ras, rad, sg):
        rr = row0 + r * _GA
        pltpu.sync_copy(src_hbm.at[pl.ds(rr, _GA)], ia)
        pltpu.sync_copy(dst_hbm.at[pl.ds(rr, _GA)], ib)
        for j in range(_GA):
            pltpu.async_copy(hwa_hbm.at[ia.at[j]],
                             ras.at[pl.ds(j * CH, CH)], sg)
            pltpu.async_copy(hwa_hbm.at[ib.at[j]],
                             rad.at[pl.ds(j * CH, CH)], sg)

    def _drain_g(ras, rad, sg):
        pltpu.make_async_copy(hwa_hbm.at[pl.ds(0, nb)], ras, sg).wait()
        pltpu.make_async_copy(hwa_hbm.at[pl.ds(0, nb)], rad, sg).wait()

    def _write(r, ras, rad, sw):
        b = (row0 + r * _GA) * CH
        pltpu.async_copy(ras, gs_hbm.at[pl.ds(b, nb)], sw)
        pltpu.async_copy(rad, gd_hbm.at[pl.ds(b, nb)], sw)

    def _drain_w(ras, rad, sw):
        pltpu.make_async_copy(ras, gs_hbm.at[pl.ds(0, nb)], sw).wait()
        pltpu.make_async_copy(rad, gd_hbm.at[pl.ds(0, nb)], sw).wait()

    _issue(0, ia0, ib0, ras0, rad0, sg0)
    _issue(1, ia1, ib1, ras1, rad1, sg1)

    @pl.loop(0, _NRA // 2)
    def _(j):
        ra = 2 * j
        rb = 2 * j + 1
        _drain_g(ras0, rad0, sg0)
        _write(ra, ras0, rad0, sw0)
        _drain_g(ras1, rad1, sg1)
        _write(rb, ras1, rad1, sw1)

        @pl.when(ra + 2 < _NRA)
        def _():
            _drain_w(ras0, rad0, sw0)
            _issue(ra + 2, ia0, ib0, ras0, rad0, sg0)

        @pl.when(rb + 2 < _NRA)
        def _():
            _drain_w(ras1, rad1, sw1)
            _issue(rb + 2, ia1, ib1, ras1, rad1, sg1)

    _drain_w(ras0, rad0, sw0)
    _drain_w(ras1, rad1, sw1)

    rt = row0 + _NRA * _GA
    pltpu.sync_copy(src_hbm.at[pl.ds(rt, 1)], ia0.at[pl.ds(0, 1)])
    pltpu.sync_copy(dst_hbm.at[pl.ds(rt, 1)], ib0.at[pl.ds(0, 1)])
    pltpu.async_copy(hwa_hbm.at[ia0.at[0]], ras0.at[pl.ds(0, CH)], sg0)
    pltpu.async_copy(hwa_hbm.at[ib0.at[0]], rad0.at[pl.ds(0, CH)], sg0)
    pltpu.make_async_copy(hwa_hbm.at[pl.ds(0, CH)],
                          ras0.at[pl.ds(0, CH)], sg0).wait()
    pltpu.make_async_copy(hwa_hbm.at[pl.ds(0, CH)],
                          rad0.at[pl.ds(0, CH)], sg0).wait()
    pltpu.sync_copy(ras0.at[pl.ds(0, CH)], gs_hbm.at[pl.ds(rt * CH, CH)])
    pltpu.sync_copy(rad0.at[pl.ds(0, CH)], gd_hbm.at[pl.ds(rt * CH, CH)])


_BE = 1600
_NBE = E // _BE


def _edge_dense_body(e_ref, gs_ref, gd_ref, we_ref, att_ref,
                     en_ref, s_ref, st_ref, acc_ref):
    i = pl.program_id(0)
    en = lax.dot_general(e_ref[...], we_ref[...], (((1,), (0,)), ((), ())),
                         precision=lax.Precision.DEFAULT,
                         preferred_element_type=jnp.float32)
    en = en + gs_ref[...] + gd_ref[...]
    en_ref[...] = en
    lr = jnp.where(en > 0, en, 0.2 * en)
    s = jnp.sum(lr * att_ref[0, :][None, :], axis=1)
    s_ref[0, 0, :] = s

    @pl.when(i == 0)
    def _():
        acc_ref[...] = jnp.zeros_like(acc_ref)

    acc_ref[0, :] += jnp.sum(en, axis=0)
    acc_ref[1, :] += jnp.sum(en * en, axis=0)

    @pl.when(i == _NBE - 1)
    def _():
        st_ref[...] = acc_ref[...]


def _edge_dense(e, gs, gd, we, att):
    return pl.pallas_call(
        _edge_dense_body,
        grid=(_NBE,),
        in_specs=[
            pl.BlockSpec((_BE, D), lambda i: (i, 0)),
            pl.BlockSpec((_BE, D), lambda i: (i, 0)),
            pl.BlockSpec((_BE, D), lambda i: (i, 0)),
            pl.BlockSpec((D, D), lambda i: (0, 0)),
            pl.BlockSpec((1, D), lambda i: (0, 0)),
        ],
        out_specs=[
            pl.BlockSpec((_BE, D), lambda i: (i, 0)),
            pl.BlockSpec((1, 1, _BE), lambda i: (i, 0, 0)),
            pl.BlockSpec((8, D), lambda i: (0, 0)),
        ],
        out_shape=[_f32((E, D)), _f32((_NBE, 1, _BE)), _f32((8, D))],
        scratch_shapes=[pltpu.VMEM((8, D), jnp.float32)],
    )(e, gs, gd, we, att)


_RT = NROWS // NS
_GC = 5
_NR2 = _RT // _GC
_DH = D // NC


@functools.partial(
    pl.kernel,
    out_type=[_f32((NC, NPAD, _DH)), _f32((NPAD,))],
    mesh=_mesh,
    compiler_params=_SC_PARAMS_NL,
    scratch_types=[
        pltpu.VMEM((_GC, CH), jnp.int32),
        pltpu.VMEM((_GC, CH), jnp.int32),
        pltpu.VMEM((_GC, CH), jnp.float32),
        pltpu.VMEM((_GC, CH), jnp.int32),
        pltpu.VMEM((_GC, CH), jnp.int32),
        pltpu.VMEM((_GC, CH), jnp.float32),
        pltpu.VMEM((10, CH), jnp.float32),
        pltpu.VMEM((10, CH), jnp.int32),
        pltpu.VMEM((_GC * CH, _DH), jnp.float32),
        pltpu.VMEM((_GC * CH, _DH), jnp.float32),
        pltpu.VMEM((_NSL,), jnp.float32),
        pltpu.VMEM_SHARED((NPAD,), jnp.float32),
        pltpu.VMEM_SHARED((NPAD, _DH), jnp.float32),
        pltpu.SemaphoreType.DMA,
        pltpu.SemaphoreType.DMA,
        pltpu.SemaphoreType.DMA,
    ],
)
def _sc_message(hwh_hbm, s_hbm, src_hbm, dst_hbm, hpart_hbm, den_hbm,
                bi0, bd0, be0, bi1, bd1, be1, sv, dv, r0, r1, zb,
                den_sh, hacc_sh, sg0, sg1, ss):
    c = lax.axis_index("c")
    sid = lax.axis_index("s")
    row0 = sid * _RT
    coff = c * N

    @pl.loop(0, _NSL, step=16)
    def _(j):
        zb[pl.ds(j, 16)] = jnp.zeros((16,), jnp.float32)

    pltpu.sync_copy(zb, den_sh.at[pl.ds(sid * _NSL, _NSL)])

    @pl.loop(0, _GC * CH)
    def _(j):
        for k in range(_DH // 16):
            r0[j, pl.ds(k * 16, 16)] = jnp.zeros((16,), jnp.float32)

    pltpu.sync_copy(r0.at[pl.ds(0, 400)],
                    hacc_sh.at[pl.ds(sid * _NSL, 400)])
    pltpu.sync_copy(r0.at[pl.ds(0, 240)],
                    hacc_sh.at[pl.ds(sid * _NSL + 400, 240)])

    plsc.subcore_barrier()

    @pl.loop(0, _RT // 10)
    def _(g):
        rr = row0 + g * 10
        pltpu.sync_copy(s_hbm.at[pl.ds(rr, 10)], sv)
        pltpu.sync_copy(dst_hbm.at[pl.ds(rr, 10)], dv)

        @pl.loop(0, 10)
        def _(j):
            for k in range(CH // 16):
                sl = pl.ds(k * 16, 16)
                sv[j, sl] = jnp.exp(sv[j, sl])

        cps = [
            pltpu.async_copy(sv.at[j], den_sh.at[dv.at[j]], ss, add=True)
            for j in range(10)
        ]
        for cp in cps:
            cp.wait()

    plsc.subcore_barrier()

    @pl.when(c == 0)
    def _():
        pltpu.sync_copy(den_sh.at[pl.ds(sid * _NSL, _NSL)], zb)
        pltpu.sync_copy(zb, den_hbm.at[pl.ds(sid * _NSL, _NSL)])

    def _issue(r, bi, bd, be, buf, sem):
        rr = row0 + r * _GC
        pltpu.sync_copy(src_hbm.at[pl.ds(rr, _GC)], bi)
        pltpu.sync_copy(dst_hbm.at[pl.ds(rr, _GC)], bd)
        pltpu.sync_copy(s_hbm.at[pl.ds(rr, _GC)], be)

        @pl.loop(0, _GC)
        def _(j):
            for k in range(CH // 16):
                sl = pl.ds(k * 16, 16)
                bi[j, sl] = bi[j, sl] + coff
                be[j, sl] = jnp.exp(be[j, sl])

        for j in range(_GC):
            pltpu.async_copy(hwh_hbm.at[bi.at[j]],
                             buf.at[pl.ds(j * CH, CH)], sem)

    def _drain(buf, sem):
        pltpu.make_async_copy(hwh_hbm.at[pl.ds(0, _GC * CH)], buf, sem).wait()

    def _scale_scatter(bd, be, buf):
        for j in range(_GC):
            @pl.loop(0, CH, unroll=4)
            def _(t):
                a16 = plsc.load_gather(be.at[j],
                                       [jnp.full((16,), t, jnp.int32)])
                for k in range(_DH // 16):
                    sl = pl.ds(k * 16, 16)
                    buf[j * CH + t, sl] = buf[j * CH + t, sl] * a16

        for j in range(_GC):
            pltpu.sync_copy(buf.at[pl.ds(j * CH, CH)],
                            hacc_sh.at[bd.at[j]], add=True)

    _issue(0, bi0, bd0, be0, r0, sg0)

    @pl.loop(0, _NR2 // 2)
    def _(i):
        ra = 2 * i
        rb = 2 * i + 1
        _issue(rb, bi1, bd1, be1, r1, sg1)
        _drain(r0, sg0)
        _scale_scatter(bd0, be0, r0)

        @pl.when(ra + 2 < _NR2)
        def _():
            _issue(ra + 2, bi0, bd0, be0, r0, sg0)

        _drain(r1, sg1)
        _scale_scatter(bd1, be1, r1)

    plsc.subcore_barrier()

    pltpu.sync_copy(hacc_sh.at[pl.ds(sid * _NSL, 400)],
                    r0.at[pl.ds(0, 400)])
    pltpu.sync_copy(r0.at[pl.ds(0, 400)],
                    hpart_hbm.at[c, pl.ds(sid * _NSL, 400)])
    pltpu.sync_copy(hacc_sh.at[pl.ds(sid * _NSL + 400, 240)],
                    r0.at[pl.ds(0, 240)])
    pltpu.sync_copy(r0.at[pl.ds(0, 240)],
                    hpart_hbm.at[c, pl.ds(sid * _NSL + 400, 240)])


def _bn_apply_body(n_rows, x_ref, xn_ref, st_ref, g_ref, b_ref, out_ref):
    mu = st_ref[0, :] / n_rows
    var = st_ref[1, :] / n_rows - mu * mu
    inv = lax.rsqrt(var + 1e-5)
    scale = g_ref[0, :] * inv
    shift = b_ref[0, :] - mu * scale
    y = xn_ref[...] * scale[None, :] + shift[None, :]
    out_ref[...] = x_ref[...] + jnp.maximum(y, 0.0)


def _bn_apply(x, xn, st, g, b, n_rows, bm):
    m = x.shape[0]
    return pl.pallas_call(
        functools.partial(_bn_apply_body, float(n_rows)),
        grid=(m // bm,),
        in_specs=[
            pl.BlockSpec((bm, D), lambda i: (i, 0)),
            pl.BlockSpec((bm, D), lambda i: (i, 0)),
            pl.BlockSpec((8, D), lambda i: (0, 0)),
            pl.BlockSpec((1, D), lambda i: (0, 0)),
            pl.BlockSpec((1, D), lambda i: (0, 0)),
        ],
        out_specs=pl.BlockSpec((bm, D), lambda i: (i, 0)),
        out_shape=_f32((m, D)),
    )(x, xn, st, g, b)


_BH = 1280
_NBH = NPAD // _BH


def _h_sum_body(hp_ref, den_ref, hn_ref, st_ref, acc_ref):
    i = pl.program_id(0)
    hn = jnp.concatenate([hp_ref[0], hp_ref[1]], axis=1)
    den = den_ref[0, 0, :]
    rden = jnp.where(den > 0, 1.0 / den, 0.0)
    hn = hn * rden[:, None]
    hn_ref[...] = hn

    @pl.when(i == 0)
    def _():
        acc_ref[...] = jnp.zeros_like(acc_ref)

    acc_ref[0, :] += jnp.sum(hn, axis=0)
    acc_ref[1, :] += jnp.sum(hn * hn, axis=0)

    @pl.when(i == _NBH - 1)
    def _():
        st_ref[...] = acc_ref[...]


def _h_sum(hpart, den):
    return pl.pallas_call(
        _h_sum_body,
        grid=(_NBH,),
        in_specs=[
            pl.BlockSpec((NC, _BH, _DH), lambda i: (0, i, 0)),
            pl.BlockSpec((1, 1, _BH), lambda i: (i, 0, 0)),
        ],
        out_specs=[
            pl.BlockSpec((_BH, D), lambda i: (i, 0)),
            pl.BlockSpec((8, D), lambda i: (0, 0)),
        ],
        out_shape=[_f32((NPAD, D)), _f32((8, D))],
        scratch_shapes=[pltpu.VMEM((8, D), jnp.float32)],
    )(hpart, den)


def _layer(src2, dst2, h, e, wh, we, wa, att, gh, bh, ge, be):
    hwa, hwh = _node_mm(h, wa, wh)
    gs, gd = _sc_gather(hwa, src2, dst2)
    en, s3, st_e = _edge_dense(e, gs, gd, we, att.reshape(1, D))
    s2 = s3.reshape(NROWS, CH)
    hpart, den = _sc_message(hwh.reshape(NC * N, _DH), s2, src2, dst2)
    e_out = _bn_apply(e, en, st_e, ge.reshape(1, D), be.reshape(1, D), E, _BE)
    hn, st_h = _h_sum(hpart, den.reshape(_NBH, 1, _BH))
    h_out = _bn_apply(h, hn[:N], st_h, gh.reshape(1, D), bh.reshape(1, D),
                      N, 1000)
    return h_out, e_out


def kernel(edge_index, h, e, Wh, We, Wa, att, gamma_h, beta_h, gamma_e,
           beta_e):
    src2 = edge_index[0].astype(jnp.int32).reshape(NROWS, CH)
    dst2 = edge_index[1].astype(jnp.int32).reshape(NROWS, CH)
    for i in range(L):
        h, e = _layer(src2, dst2, h, e, Wh[i], We[i], Wa[i],
                      att[i], gamma_h[i], beta_h[i], gamma_e[i], beta_e[i])
    return (h, e)


# speedup vs baseline: 6.2825x; 1.0854x over previous
import functools

import jax
import jax.numpy as jnp
from jax import lax
from jax.experimental import pallas as pl
from jax.experimental.pallas import tpu as pltpu
from jax.experimental.pallas import tpu_sc as plsc

N = 10000
E = 320000
D = 128
L = 2
NPAD = 10240

NC = 2
NS = 16
NW = NC * NS
EW = E // NW
CH = 80
NCHW = EW // CH
NROWS = E // CH
_NSL = NPAD // NS

_PREC = lax.Precision.HIGHEST

_mesh = plsc.VectorSubcoreMesh(core_axis_name="c", subcore_axis_name="s")
_SC_PARAMS = pltpu.CompilerParams(use_tc_tiling_on_sc=False)
_SC_PARAMS_NL = pltpu.CompilerParams(use_tc_tiling_on_sc=False,
                                     needs_layout_passes=False)


def _f32(shape):
    return jax.ShapeDtypeStruct(shape, jnp.float32)


def _node_mm_body(h_ref, wa_ref, wh_ref, hwa_ref, hwh2_ref):
    h = h_ref[...]
    hwa_ref[...] = lax.dot_general(h, wa_ref[...], (((1,), (0,)), ((), ())),
                                   precision=_PREC,
                                   preferred_element_type=jnp.float32)
    hwh = lax.dot_general(h, wh_ref[...], (((1,), (0,)), ((), ())),
                          precision=_PREC,
                          preferred_element_type=jnp.float32)
    hwh2_ref[0] = hwh[:, :64]
    hwh2_ref[1] = hwh[:, 64:]


def _node_mm(h, wa, wh):
    bn = 2000
    return pl.pallas_call(
        _node_mm_body,
        grid=(N // bn,),
        in_specs=[
            pl.BlockSpec((bn, D), lambda i: (i, 0)),
            pl.BlockSpec((D, D), lambda i: (0, 0)),
            pl.BlockSpec((D, D), lambda i: (0, 0)),
        ],
        out_specs=[
            pl.BlockSpec((bn, D), lambda i: (i, 0)),
            pl.BlockSpec((NC, bn, 64), lambda i: (0, i, 0)),
        ],
        out_shape=[_f32((N, D)), _f32((NC, N, 64))],
    )(h, wa, wh)


_GA = 2
_NRA = 62


@functools.partial(
    pl.kernel,
    out_type=[_f32((E, D)), _f32((E, D))],
    mesh=_mesh,
    compiler_params=_SC_PARAMS,
    scratch_types=[
        pltpu.VMEM((_GA, CH), jnp.int32),
        pltpu.VMEM((_GA, CH), jnp.int32),
        pltpu.VMEM((_GA, CH), jnp.int32),
        pltpu.VMEM((_GA, CH), jnp.int32),
        pltpu.VMEM((_GA * CH, D), jnp.float32),
        pltpu.VMEM((_GA * CH, D), jnp.float32),
        pltpu.VMEM((_GA * CH, D), jnp.float32),
        pltpu.VMEM((_GA * CH, D), jnp.float32),
        pltpu.SemaphoreType.DMA,
        pltpu.SemaphoreType.DMA,
        pltpu.SemaphoreType.DMA,
        pltpu.SemaphoreType.DMA,
    ],
)
def _sc_gather(hwa_hbm, src_hbm, dst_hbm, gs_hbm, gd_hbm,
               ia0, ib0, ia1, ib1, ras0, rad0, ras1, rad1,
               sg0, sg1, sw0, sw1):
    wid = lax.axis_index("s") * NC + lax.axis_index("c")
    row0 = wid * NCHW
    nb = _GA * CH

    def _issue(r, ia, ib, ras, rad, sg):
        rr = row0 + r * _GA
        pltpu.sync_copy(src_hbm.at[pl.ds(rr, _GA)], ia)
        pltpu.sync_copy(dst_hbm.at[pl.ds(rr, _GA)], ib)
        for j in range(_GA):
            pltpu.async_copy(hwa_hbm.at[ia.at[j]],
                             ras.at[pl.ds(j * CH, CH)], sg)
            pltpu.async_copy(hwa_hbm.at[ib.at[j]],
                             rad.at[pl.ds(j * CH, CH)], sg)

    def _drain_g(ras, rad, sg):
        pltpu.make_async_copy(hwa_hbm.at[pl.ds(0, nb)], ras, sg).wait()
        pltpu.make_async_copy(hwa_hbm.at[pl.ds(0, nb)], rad, sg).wait()

    def _write(r, ras, rad, sw):
        b = (row0 + r * _GA) * CH
        pltpu.async_copy(ras, gs_hbm.at[pl.ds(b, nb)], sw)
        pltpu.async_copy(rad, gd_hbm.at[pl.ds(b, nb)], sw)

    def _drain_w(ras, rad, sw):
        pltpu.make_async_copy(ras, gs_hbm.at[pl.ds(0, nb)], sw).wait()
        pltpu.make_async_copy(rad, gd_hbm.at[pl.ds(0, nb)], sw).wait()

    _issue(0, ia0, ib0, ras0, rad0, sg0)
    _issue(1, ia1, ib1, ras1, rad1, sg1)

    @pl.loop(0, _NRA // 2)
    def _(j):
        ra = 2 * j
        rb = 2 * j + 1
        _drain_g(ras0, rad0, sg0)
        _write(ra, ras0, rad0, sw0)
        _drain_g(ras1, rad1, sg1)
        _write(rb, ras1, rad1, sw1)

        @pl.when(ra + 2 < _NRA)
        def _():
            _drain_w(ras0, rad0, sw0)
            _issue(ra + 2, ia0, ib0, ras0, rad0, sg0)

        @pl.when(rb + 2 < _NRA)
        def _():
            _drain_w(ras1, rad1, sw1)
            _issue(rb + 2, ia1, ib1, ras1, rad1, sg1)

    _drain_w(ras0, rad0, sw0)
    _drain_w(ras1, rad1, sw1)

    rt = row0 + _NRA * _GA
    pltpu.sync_copy(src_hbm.at[pl.ds(rt, 1)], ia0.at[pl.ds(0, 1)])
    pltpu.sync_copy(dst_hbm.at[pl.ds(rt, 1)], ib0.at[pl.ds(0, 1)])
    pltpu.async_copy(hwa_hbm.at[ia0.at[0]], ras0.at[pl.ds(0, CH)], sg0)
    pltpu.async_copy(hwa_hbm.at[ib0.at[0]], rad0.at[pl.ds(0, CH)], sg0)
    pltpu.make_async_copy(hwa_hbm.at[pl.ds(0, CH)],
                          ras0.at[pl.ds(0, CH)], sg0).wait()
    pltpu.make_async_copy(hwa_hbm.at[pl.ds(0, CH)],
                          rad0.at[pl.ds(0, CH)], sg0).wait()
    pltpu.sync_copy(ras0.at[pl.ds(0, CH)], gs_hbm.at[pl.ds(rt * CH, CH)])
    pltpu.sync_copy(rad0.at[pl.ds(0, CH)], gd_hbm.at[pl.ds(rt * CH, CH)])


_BE = 1600
_NBE = E // _BE


def _edge_dense_body(e_ref, gs_ref, gd_ref, we_ref, att_ref,
                     en_ref, s_ref, st_ref, acc_ref):
    i = pl.program_id(0)
    en = lax.dot_general(e_ref[...], we_ref[...], (((1,), (0,)), ((), ())),
                         precision=lax.Precision.DEFAULT,
                         preferred_element_type=jnp.float32)
    en = en + gs_ref[...].astype(jnp.float32) + gd_ref[...].astype(jnp.float32)
    en_ref[...] = en.astype(jnp.bfloat16)
    lr = jnp.where(en > 0, en, 0.2 * en)
    s = jnp.sum(lr * att_ref[0, :][None, :], axis=1)
    s_ref[0, 0, :] = s

    @pl.when(i == 0)
    def _():
        acc_ref[...] = jnp.zeros_like(acc_ref)

    acc_ref[0, :] += jnp.sum(en, axis=0)
    acc_ref[1, :] += jnp.sum(en * en, axis=0)

    @pl.when(i == _NBE - 1)
    def _():
        st_ref[...] = acc_ref[...]


def _edge_dense(e, gs, gd, we, att):
    return pl.pallas_call(
        _edge_dense_body,
        grid=(_NBE,),
        in_specs=[
            pl.BlockSpec((_BE, D), lambda i: (i, 0)),
            pl.BlockSpec((_BE, D), lambda i: (i, 0)),
            pl.BlockSpec((_BE, D), lambda i: (i, 0)),
            pl.BlockSpec((D, D), lambda i: (0, 0)),
            pl.BlockSpec((1, D), lambda i: (0, 0)),
        ],
        out_specs=[
            pl.BlockSpec((_BE, D), lambda i: (i, 0)),
            pl.BlockSpec((1, 1, _BE), lambda i: (i, 0, 0)),
            pl.BlockSpec((8, D), lambda i: (0, 0)),
        ],
        out_shape=[jax.ShapeDtypeStruct((E, D), jnp.bfloat16),
                   _f32((_NBE, 1, _BE)), _f32((8, D))],
        scratch_shapes=[pltpu.VMEM((8, D), jnp.float32)],
    )(e, gs, gd, we, att)


_RT = NROWS // NS
_GC = 5
_NR2 = _RT // _GC
_DH = D // NC


@functools.partial(
    pl.kernel,
    out_type=[_f32((NC, NPAD, _DH)), _f32((NPAD,))],
    mesh=_mesh,
    compiler_params=_SC_PARAMS_NL,
    scratch_types=[
        pltpu.VMEM((_GC, CH), jnp.int32),
        pltpu.VMEM((_GC, CH), jnp.int32),
        pltpu.VMEM((_GC, CH), jnp.float32),
        pltpu.VMEM((_GC, CH), jnp.int32),
        pltpu.VMEM((_GC, CH), jnp.int32),
        pltpu.VMEM((_GC, CH), jnp.float32),
        pltpu.VMEM((_GC * CH, _DH), jnp.float32),
        pltpu.VMEM((_GC * CH, _DH), jnp.float32),
        pltpu.VMEM((_NSL,), jnp.float32),
        pltpu.VMEM_SHARED((NPAD,), jnp.float32),
        pltpu.VMEM_SHARED((NPAD, _DH), jnp.float32),
        pltpu.SemaphoreType.DMA,
        pltpu.SemaphoreType.DMA,
        pltpu.SemaphoreType.DMA,
        pltpu.SemaphoreType.DMA,
    ],
)
def _sc_message(hwh_hbm, s_hbm, src_hbm, dst_hbm, hpart_hbm, den_hbm,
                bi0, bd0, be0, bi1, bd1, be1, r0, r1, zb,
                den_sh, hacc_sh, sg0, sg1, ss0, ss1):
    c = lax.axis_index("c")
    sid = lax.axis_index("s")
    row0 = sid * _RT
    coff = c * N

    @pl.loop(0, _NSL, step=16)
    def _(j):
        zb[pl.ds(j, 16)] = jnp.zeros((16,), jnp.float32)

    pltpu.sync_copy(zb, den_sh.at[pl.ds(sid * _NSL, _NSL)])

    @pl.loop(0, _GC * CH)
    def _(j):
        for k in range(_DH // 16):
            r0[j, pl.ds(k * 16, 16)] = jnp.zeros((16,), jnp.float32)

    pltpu.sync_copy(r0.at[pl.ds(0, 400)],
                    hacc_sh.at[pl.ds(sid * _NSL, 400)])
    pltpu.sync_copy(r0.at[pl.ds(0, 240)],
                    hacc_sh.at[pl.ds(sid * _NSL + 400, 240)])

    plsc.subcore_barrier()

    def _drain_scat(bd, be, buf, ss):
        for j in range(_GC):
            pltpu.make_async_copy(buf.at[pl.ds(j * CH, CH)],
                                  hacc_sh.at[bd.at[j]], ss).wait()
            pltpu.make_async_copy(be.at[j], den_sh.at[bd.at[j]], ss).wait()

    def _issue(r, bi, bd, be, buf, sg, ss, first):
        if not first:
            _drain_scat(bd, be, buf, ss)
        rr = row0 + r * _GC
        pltpu.sync_copy(src_hbm.at[pl.ds(rr, _GC)], bi)
        pltpu.sync_copy(dst_hbm.at[pl.ds(rr, _GC)], bd)
        pltpu.sync_copy(s_hbm.at[pl.ds(rr, _GC)], be)

        @pl.loop(0, _GC)
        def _(j):
            for k in range(CH // 16):
                sl = pl.ds(k * 16, 16)
                bi[j, sl] = bi[j, sl] + coff
                be[j, sl] = jnp.exp(be[j, sl])

        for j in range(_GC):
            pltpu.async_copy(hwh_hbm.at[bi.at[j]],
                             buf.at[pl.ds(j * CH, CH)], sg)

    def _drain_g(buf, sg):
        pltpu.make_async_copy(hwh_hbm.at[pl.ds(0, _GC * CH)], buf, sg).wait()

    def _scale_scatter(bd, be, buf, ss):
        for j in range(_GC):
            @pl.loop(0, CH, unroll=4)
            def _(t):
                a16 = plsc.load_gather(be.at[j],
                                       [jnp.full((16,), t, jnp.int32)])
                for k in range(_DH // 16):
                    sl = pl.ds(k * 16, 16)
                    buf[j * CH + t, sl] = buf[j * CH + t, sl] * a16

        for j in range(_GC):
            pltpu.async_copy(buf.at[pl.ds(j * CH, CH)],
                             hacc_sh.at[bd.at[j]], ss, add=True)
            pltpu.async_copy(be.at[j], den_sh.at[bd.at[j]], ss, add=True)

    _issue(0, bi0, bd0, be0, r0, sg0, ss0, True)

    @pl.loop(0, _NR2 // 2)
    def _(i):
        ra = 2 * i
        rb = 2 * i + 1

        @pl.when(i == 0)
        def _():
            _issue(rb, bi1, bd1, be1, r1, sg1, ss1, True)

        @pl.when(i > 0)
        def _():
            _issue(rb, bi1, bd1, be1, r1, sg1, ss1, False)

        _drain_g(r0, sg0)
        _scale_scatter(bd0, be0, r0, ss0)

        @pl.when(ra + 2 < _NR2)
        def _():
            _issue(ra + 2, bi0, bd0, be0, r0, sg0, ss0, False)

        _drain_g(r1, sg1)
        _scale_scatter(bd1, be1, r1, ss1)

    _drain_scat(bd0, be0, r0, ss0)
    _drain_scat(bd1, be1, r1, ss1)
    plsc.subcore_barrier()

    @pl.when(c == 0)
    def _():
        pltpu.sync_copy(den_sh.at[pl.ds(sid * _NSL, _NSL)], zb)
        pltpu.sync_copy(zb, den_hbm.at[pl.ds(sid * _NSL, _NSL)])

    pltpu.sync_copy(hacc_sh.at[pl.ds(sid * _NSL, 400)],
                    r0.at[pl.ds(0, 400)])
    pltpu.sync_copy(r0.at[pl.ds(0, 400)],
                    hpart_hbm.at[c, pl.ds(sid * _NSL, 400)])
    pltpu.sync_copy(hacc_sh.at[pl.ds(sid * _NSL + 400, 240)],
                    r0.at[pl.ds(0, 240)])
    pltpu.sync_copy(r0.at[pl.ds(0, 240)],
                    hpart_hbm.at[c, pl.ds(sid * _NSL + 400, 240)])


def _bn_apply_body(n_rows, x_ref, xn_ref, st_ref, g_ref, b_ref, out_ref):
    mu = st_ref[0, :] / n_rows
    var = st_ref[1, :] / n_rows - mu * mu
    inv = lax.rsqrt(var + 1e-5)
    scale = g_ref[0, :] * inv
    shift = b_ref[0, :] - mu * scale
    y = xn_ref[...].astype(jnp.float32) * scale[None, :] + shift[None, :]
    out_ref[...] = x_ref[...] + jnp.maximum(y, 0.0)


def _bn_apply(x, xn, st, g, b, n_rows, bm):
    m = x.shape[0]
    return pl.pallas_call(
        functools.partial(_bn_apply_body, float(n_rows)),
        grid=(m // bm,),
        in_specs=[
            pl.BlockSpec((bm, D), lambda i: (i, 0)),
            pl.BlockSpec((bm, D), lambda i: (i, 0)),
            pl.BlockSpec((8, D), lambda i: (0, 0)),
            pl.BlockSpec((1, D), lambda i: (0, 0)),
            pl.BlockSpec((1, D), lambda i: (0, 0)),
        ],
        out_specs=pl.BlockSpec((bm, D), lambda i: (i, 0)),
        out_shape=_f32((m, D)),
    )(x, xn, st, g, b)


_BH = 1280
_NBH = NPAD // _BH


def _h_sum_body(hp_ref, den_ref, hn_ref, st_ref, acc_ref):
    i = pl.program_id(0)
    hn = jnp.concatenate([hp_ref[0], hp_ref[1]], axis=1)
    den = den_ref[0, 0, :]
    rden = jnp.where(den > 0, 1.0 / den, 0.0)
    hn = hn * rden[:, None]
    hn_ref[...] = hn

    @pl.when(i == 0)
    def _():
        acc_ref[...] = jnp.zeros_like(acc_ref)

    acc_ref[0, :] += jnp.sum(hn, axis=0)
    acc_ref[1, :] += jnp.sum(hn * hn, axis=0)

    @pl.when(i == _NBH - 1)
    def _():
        st_ref[...] = acc_ref[...]


def _h_sum(hpart, den):
    return pl.pallas_call(
        _h_sum_body,
        grid=(_NBH,),
        in_specs=[
            pl.BlockSpec((NC, _BH, _DH), lambda i: (0, i, 0)),
            pl.BlockSpec((1, 1, _BH), lambda i: (i, 0, 0)),
        ],
        out_specs=[
            pl.BlockSpec((_BH, D), lambda i: (i, 0)),
            pl.BlockSpec((8, D), lambda i: (0, 0)),
        ],
        out_shape=[_f32((NPAD, D)), _f32((8, D))],
        scratch_shapes=[pltpu.VMEM((8, D), jnp.float32)],
    )(hpart, den)


def _layer(src2, dst2, h, e, wh, we, wa, att, gh, bh, ge, be):
    hwa, hwh = _node_mm(h, wa, wh)
    gs, gd = _sc_gather(hwa, src2, dst2)
    en, s3, st_e = _edge_dense(e, gs, gd, we, att.reshape(1, D))
    s2 = s3.reshape(NROWS, CH)
    hpart, den = _sc_message(hwh.reshape(NC * N, _DH), s2, src2, dst2)
    e_out = _bn_apply(e, en, st_e, ge.reshape(1, D), be.reshape(1, D), E, _BE)
    hn, st_h = _h_sum(hpart, den.reshape(_NBH, 1, _BH))
    h_out = _bn_apply(h, hn[:N], st_h, gh.reshape(1, D), bh.reshape(1, D),
                      N, 1000)
    return h_out, e_out


def kernel(edge_index, h, e, Wh, We, Wa, att, gamma_h, beta_h, gamma_e,
           beta_e):
    src2 = edge_index[0].astype(jnp.int32).reshape(NROWS, CH)
    dst2 = edge_index[1].astype(jnp.int32).reshape(NROWS, CH)
    for i in range(L):
        h, e = _layer(src2, dst2, h, e, Wh[i], We[i], Wa[i],
                      att[i], gamma_h[i], beta_h[i], gamma_e[i], beta_e[i])
    return (h, e)


# speedup vs baseline: 6.4793x; 1.0313x over previous
import functools

import jax
import jax.numpy as jnp
from jax import lax
from jax.experimental import pallas as pl
from jax.experimental.pallas import tpu as pltpu
from jax.experimental.pallas import tpu_sc as plsc

N = 10000
E = 320000
D = 128
L = 2
NPAD = 10240

NC = 2
NS = 16
NW = NC * NS
EW = E // NW
CH = 80
NCHW = EW // CH
NROWS = E // CH
_NSL = NPAD // NS

_PREC = lax.Precision.HIGHEST

_mesh = plsc.VectorSubcoreMesh(core_axis_name="c", subcore_axis_name="s")
_SC_PARAMS = pltpu.CompilerParams(use_tc_tiling_on_sc=False)
_SC_PARAMS_NL = pltpu.CompilerParams(use_tc_tiling_on_sc=False,
                                     needs_layout_passes=False)


def _f32(shape):
    return jax.ShapeDtypeStruct(shape, jnp.float32)


def _node_mm_body(h_ref, wa_ref, wh_ref, hwa_ref, hwh2_ref):
    h = h_ref[...]
    hwa_ref[...] = lax.dot_general(h, wa_ref[...], (((1,), (0,)), ((), ())),
                                   precision=_PREC,
                                   preferred_element_type=jnp.float32)
    hwh = lax.dot_general(h, wh_ref[...], (((1,), (0,)), ((), ())),
                          precision=_PREC,
                          preferred_element_type=jnp.float32)
    hwh2_ref[0] = hwh[:, :64]
    hwh2_ref[1] = hwh[:, 64:]


def _node_mm(h, wa, wh):
    bn = 2000
    return pl.pallas_call(
        _node_mm_body,
        grid=(N // bn,),
        in_specs=[
            pl.BlockSpec((bn, D), lambda i: (i, 0)),
            pl.BlockSpec((D, D), lambda i: (0, 0)),
            pl.BlockSpec((D, D), lambda i: (0, 0)),
        ],
        out_specs=[
            pl.BlockSpec((bn, D), lambda i: (i, 0)),
            pl.BlockSpec((NC, bn, 64), lambda i: (0, i, 0)),
        ],
        out_shape=[_f32((N, D)), _f32((NC, N, 64))],
    )(h, wa, wh)


_GA = 2
_NRA = 62


@functools.partial(
    pl.kernel,
    out_type=[_f32((E, D)), _f32((E, D))],
    mesh=_mesh,
    compiler_params=_SC_PARAMS,
    scratch_types=[
        pltpu.VMEM((2, _GA, CH), jnp.int32),
        pltpu.VMEM((2, _GA, CH), jnp.int32),
        pltpu.VMEM((_GA * CH, D), jnp.float32),
        pltpu.VMEM((_GA * CH, D), jnp.float32),
        pltpu.VMEM((_GA * CH, D), jnp.float32),
        pltpu.VMEM((_GA * CH, D), jnp.float32),
        pltpu.SemaphoreType.DMA,
        pltpu.SemaphoreType.DMA,
        pltpu.SemaphoreType.DMA,
        pltpu.SemaphoreType.DMA,
    ],
)
def _sc_gather(hwa_hbm, ei_hbm, gs_hbm, gd_hbm,
               iab0, iab1, ras0, rad0, ras1, rad1,
               sg0, sg1, sw0, sw1):
    wid = lax.axis_index("s") * NC + lax.axis_index("c")
    row0 = wid * NCHW
    nb = _GA * CH

    def _issue(r, iab, ras, rad, sg):
        rr = row0 + r * _GA
        pltpu.sync_copy(ei_hbm.at[:, pl.ds(rr, _GA)], iab)
        for j in range(_GA):
            pltpu.async_copy(hwa_hbm.at[iab.at[0, j]],
                             ras.at[pl.ds(j * CH, CH)], sg)
            pltpu.async_copy(hwa_hbm.at[iab.at[1, j]],
                             rad.at[pl.ds(j * CH, CH)], sg)

    def _drain_g(ras, rad, sg):
        pltpu.make_async_copy(hwa_hbm.at[pl.ds(0, nb)], ras, sg).wait()
        pltpu.make_async_copy(hwa_hbm.at[pl.ds(0, nb)], rad, sg).wait()

    def _write(r, ras, rad, sw):
        b = (row0 + r * _GA) * CH
        pltpu.async_copy(ras, gs_hbm.at[pl.ds(b, nb)], sw)
        pltpu.async_copy(rad, gd_hbm.at[pl.ds(b, nb)], sw)

    def _drain_w(ras, rad, sw):
        pltpu.make_async_copy(ras, gs_hbm.at[pl.ds(0, nb)], sw).wait()
        pltpu.make_async_copy(rad, gd_hbm.at[pl.ds(0, nb)], sw).wait()

    _issue(0, iab0, ras0, rad0, sg0)
    _issue(1, iab1, ras1, rad1, sg1)

    @pl.loop(0, _NRA // 2)
    def _(j):
        ra = 2 * j
        rb = 2 * j + 1
        _drain_g(ras0, rad0, sg0)
        _write(ra, ras0, rad0, sw0)
        _drain_g(ras1, rad1, sg1)
        _write(rb, ras1, rad1, sw1)

        @pl.when(ra + 2 < _NRA)
        def _():
            _drain_w(ras0, rad0, sw0)
            _issue(ra + 2, iab0, ras0, rad0, sg0)

        @pl.when(rb + 2 < _NRA)
        def _():
            _drain_w(ras1, rad1, sw1)
            _issue(rb + 2, iab1, ras1, rad1, sg1)

    _drain_w(ras0, rad0, sw0)
    _drain_w(ras1, rad1, sw1)

    rt = row0 + _NRA * _GA
    pltpu.sync_copy(ei_hbm.at[:, pl.ds(rt, 1)], iab0.at[:, pl.ds(0, 1)])
    pltpu.async_copy(hwa_hbm.at[iab0.at[0, 0]], ras0.at[pl.ds(0, CH)], sg0)
    pltpu.async_copy(hwa_hbm.at[iab0.at[1, 0]], rad0.at[pl.ds(0, CH)], sg0)
    pltpu.make_async_copy(hwa_hbm.at[pl.ds(0, CH)],
                          ras0.at[pl.ds(0, CH)], sg0).wait()
    pltpu.make_async_copy(hwa_hbm.at[pl.ds(0, CH)],
                          rad0.at[pl.ds(0, CH)], sg0).wait()
    pltpu.sync_copy(ras0.at[pl.ds(0, CH)], gs_hbm.at[pl.ds(rt * CH, CH)])
    pltpu.sync_copy(rad0.at[pl.ds(0, CH)], gd_hbm.at[pl.ds(rt * CH, CH)])


_BE = 1600
_NBE = E // _BE


def _edge_dense_body(e_ref, gs_ref, gd_ref, we_ref, att_ref,
                     en_ref, s_ref, st_ref, acc_ref):
    i = pl.program_id(0)
    en = lax.dot_general(e_ref[...], we_ref[...], (((1,), (0,)), ((), ())),
                         precision=lax.Precision.DEFAULT,
                         preferred_element_type=jnp.float32)
    en = en + gs_ref[...].astype(jnp.float32) + gd_ref[...].astype(jnp.float32)
    en_ref[...] = en.astype(jnp.bfloat16)
    lr = jnp.where(en > 0, en, 0.2 * en)
    s = lax.dot_general(lr, att_ref[...], (((1,), (1,)), ((), ())),
                        precision=lax.Precision.DEFAULT,
                        preferred_element_type=jnp.float32)
    s_ref[0, 0, :] = s[:, 0]

    @pl.when(i == 0)
    def _():
        acc_ref[...] = jnp.zeros_like(acc_ref)

    acc_ref[0, :] += jnp.sum(en, axis=0)
    acc_ref[1, :] += jnp.sum(en * en, axis=0)

    @pl.when(i == _NBE - 1)
    def _():
        st_ref[...] = acc_ref[...]


def _edge_dense(e, gs, gd, we, att):
    return pl.pallas_call(
        _edge_dense_body,
        grid=(_NBE,),
        in_specs=[
            pl.BlockSpec((_BE, D), lambda i: (i, 0)),
            pl.BlockSpec((_BE, D), lambda i: (i, 0)),
            pl.BlockSpec((_BE, D), lambda i: (i, 0)),
            pl.BlockSpec((D, D), lambda i: (0, 0)),
            pl.BlockSpec((1, D), lambda i: (0, 0)),
        ],
        out_specs=[
            pl.BlockSpec((_BE, D), lambda i: (i, 0)),
            pl.BlockSpec((1, 1, _BE), lambda i: (i, 0, 0)),
            pl.BlockSpec((8, D), lambda i: (0, 0)),
        ],
        out_shape=[jax.ShapeDtypeStruct((E, D), jnp.bfloat16),
                   _f32((_NBE, 1, _BE)), _f32((8, D))],
        scratch_shapes=[pltpu.VMEM((8, D), jnp.float32)],
    )(e, gs, gd, we, att)


_RT = NROWS // NS
_GC = 5
_NR2 = _RT // _GC
_DH = D // NC


@functools.partial(
    pl.kernel,
    out_type=[_f32((NC, NPAD, _DH)), _f32((NPAD,))],
    mesh=_mesh,
    compiler_params=_SC_PARAMS_NL,
    scratch_types=[
        pltpu.VMEM((2, _GC, CH), jnp.int32),
        pltpu.VMEM((_GC, CH), jnp.float32),
        pltpu.VMEM((2, _GC, CH), jnp.int32),
        pltpu.VMEM((_GC, CH), jnp.float32),
        pltpu.VMEM((_GC * CH, _DH), jnp.float32),
        pltpu.VMEM((_GC * CH, _DH), jnp.float32),
        pltpu.VMEM((_NSL,), jnp.float32),
        pltpu.VMEM_SHARED((NPAD,), jnp.float32),
        pltpu.VMEM_SHARED((NPAD, _DH), jnp.float32),
        pltpu.SemaphoreType.DMA,
        pltpu.SemaphoreType.DMA,
        pltpu.SemaphoreType.DMA,
        pltpu.SemaphoreType.DMA,
    ],
)
def _sc_message(hwh_hbm, s_hbm, ei_hbm, hpart_hbm, den_hbm,
                bid0, be0, bid1, be1, r0, r1, zb,
                den_sh, hacc_sh, sg0, sg1, ss0, ss1):
    c = lax.axis_index("c")
    sid = lax.axis_index("s")
    row0 = sid * _RT
    coff = c * N

    @pl.loop(0, _NSL, step=16)
    def _(j):
        zb[pl.ds(j, 16)] = jnp.zeros((16,), jnp.float32)

    pltpu.sync_copy(zb, den_sh.at[pl.ds(sid * _NSL, _NSL)])

    @pl.loop(0, _GC * CH)
    def _(j):
        for k in range(_DH // 16):
            r0[j, pl.ds(k * 16, 16)] = jnp.zeros((16,), jnp.float32)

    pltpu.sync_copy(r0.at[pl.ds(0, 400)],
                    hacc_sh.at[pl.ds(sid * _NSL, 400)])
    pltpu.sync_copy(r0.at[pl.ds(0, 240)],
                    hacc_sh.at[pl.ds(sid * _NSL + 400, 240)])

    plsc.subcore_barrier()

    def _drain_scat(bid, be, buf, ss):
        for j in range(_GC):
            pltpu.make_async_copy(buf.at[pl.ds(j * CH, CH)],
                                  hacc_sh.at[bid.at[1, j]], ss).wait()
            pltpu.make_async_copy(be.at[j], den_sh.at[bid.at[1, j]], ss).wait()

    def _issue(r, bid, be, buf, sg, ss, first):
        if not first:
            _drain_scat(bid, be, buf, ss)
        rr = row0 + r * _GC
        pltpu.sync_copy(ei_hbm.at[:, pl.ds(rr, _GC)], bid)
        pltpu.sync_copy(s_hbm.at[pl.ds(rr, _GC)], be)

        @pl.loop(0, _GC)
        def _(j):
            for k in range(CH // 16):
                sl = pl.ds(k * 16, 16)
                bid[0, j, sl] = bid[0, j, sl] + coff
                be[j, sl] = jnp.exp(be[j, sl])

        for j in range(_GC):
            pltpu.async_copy(hwh_hbm.at[bid.at[0, j]],
                             buf.at[pl.ds(j * CH, CH)], sg)

    def _drain_g(buf, sg):
        pltpu.make_async_copy(hwh_hbm.at[pl.ds(0, _GC * CH)], buf, sg).wait()

    def _scale_scatter(bid, be, buf, ss):
        for j in range(_GC):
            @pl.loop(0, CH, unroll=4)
            def _(t):
                a16 = plsc.load_gather(be.at[j],
                                       [jnp.full((16,), t, jnp.int32)])
                for k in range(_DH // 16):
                    sl = pl.ds(k * 16, 16)
                    buf[j * CH + t, sl] = buf[j * CH + t, sl] * a16

        for j in range(_GC):
            pltpu.async_copy(buf.at[pl.ds(j * CH, CH)],
                             hacc_sh.at[bid.at[1, j]], ss, add=True)
            pltpu.async_copy(be.at[j], den_sh.at[bid.at[1, j]], ss, add=True)

    _issue(0, bid0, be0, r0, sg0, ss0, True)

    @pl.loop(0, _NR2 // 2)
    def _(i):
        ra = 2 * i
        rb = 2 * i + 1

        @pl.when(i == 0)
        def _():
            _issue(rb, bid1, be1, r1, sg1, ss1, True)

        @pl.when(i > 0)
        def _():
            _issue(rb, bid1, be1, r1, sg1, ss1, False)

        _drain_g(r0, sg0)
        _scale_scatter(bid0, be0, r0, ss0)

        @pl.when(ra + 2 < _NR2)
        def _():
            _issue(ra + 2, bid0, be0, r0, sg0, ss0, False)

        _drain_g(r1, sg1)
        _scale_scatter(bid1, be1, r1, ss1)

    _drain_scat(bid0, be0, r0, ss0)
    _drain_scat(bid1, be1, r1, ss1)
    plsc.subcore_barrier()

    @pl.when(c == 0)
    def _():
        pltpu.sync_copy(den_sh.at[pl.ds(sid * _NSL, _NSL)], zb)
        pltpu.sync_copy(zb, den_hbm.at[pl.ds(sid * _NSL, _NSL)])

    pltpu.sync_copy(hacc_sh.at[pl.ds(sid * _NSL, 400)],
                    r0.at[pl.ds(0, 400)])
    pltpu.sync_copy(r0.at[pl.ds(0, 400)],
                    hpart_hbm.at[c, pl.ds(sid * _NSL, 400)])
    pltpu.sync_copy(hacc_sh.at[pl.ds(sid * _NSL + 400, 240)],
                    r0.at[pl.ds(0, 240)])
    pltpu.sync_copy(r0.at[pl.ds(0, 240)],
                    hpart_hbm.at[c, pl.ds(sid * _NSL + 400, 240)])


def _bn_apply_body(n_rows, x_ref, xn_ref, st_ref, g_ref, b_ref, out_ref):
    mu = st_ref[0, :] / n_rows
    var = st_ref[1, :] / n_rows - mu * mu
    inv = lax.rsqrt(var + 1e-5)
    scale = g_ref[0, :] * inv
    shift = b_ref[0, :] - mu * scale
    y = xn_ref[...].astype(jnp.float32) * scale[None, :] + shift[None, :]
    out_ref[...] = x_ref[...] + jnp.maximum(y, 0.0)


def _bn_apply(x, xn, st, g, b, n_rows, bm):
    m = x.shape[0]
    return pl.pallas_call(
        functools.partial(_bn_apply_body, float(n_rows)),
        grid=(m // bm,),
        in_specs=[
            pl.BlockSpec((bm, D), lambda i: (i, 0)),
            pl.BlockSpec((bm, D), lambda i: (i, 0)),
            pl.BlockSpec((8, D), lambda i: (0, 0)),
            pl.BlockSpec((1, D), lambda i: (0, 0)),
            pl.BlockSpec((1, D), lambda i: (0, 0)),
        ],
        out_specs=pl.BlockSpec((bm, D), lambda i: (i, 0)),
        out_shape=_f32((m, D)),
    )(x, xn, st, g, b)


_BH = 1280
_NBH = NPAD // _BH


def _h_sum_body(hp_ref, den_ref, hn_ref, st_ref, acc_ref):
    i = pl.program_id(0)
    hn = jnp.concatenate([hp_ref[0], hp_ref[1]], axis=1)
    den = den_ref[0, 0, :]
    rden = jnp.where(den > 0, 1.0 / den, 0.0)
    hn = hn * rden[:, None]
    hn_ref[...] = hn

    @pl.when(i == 0)
    def _():
        acc_ref[...] = jnp.zeros_like(acc_ref)

    acc_ref[0, :] += jnp.sum(hn, axis=0)
    acc_ref[1, :] += jnp.sum(hn * hn, axis=0)

    @pl.when(i == _NBH - 1)
    def _():
        st_ref[...] = acc_ref[...]


def _h_sum(hpart, den):
    return pl.pallas_call(
        _h_sum_body,
        grid=(_NBH,),
        in_specs=[
            pl.BlockSpec((NC, _BH, _DH), lambda i: (0, i, 0)),
            pl.BlockSpec((1, 1, _BH), lambda i: (i, 0, 0)),
        ],
        out_specs=[
            pl.BlockSpec((_BH, D), lambda i: (i, 0)),
            pl.BlockSpec((8, D), lambda i: (0, 0)),
        ],
        out_shape=[_f32((NPAD, D)), _f32((8, D))],
        scratch_shapes=[pltpu.VMEM((8, D), jnp.float32)],
    )(hpart, den)


def _layer(ei2, h, e, wh, we, wa, att, gh, bh, ge, be):
    hwa, hwh = _node_mm(h, wa, wh)
    gs, gd = _sc_gather(hwa, ei2)
    en, s3, st_e = _edge_dense(e, gs, gd, we, att.reshape(1, D))
    s2 = s3.reshape(NROWS, CH)
    hpart, den = _sc_message(hwh.reshape(NC * N, _DH), s2, ei2)
    e_out = _bn_apply(e, en, st_e, ge.reshape(1, D), be.reshape(1, D), E, _BE)
    hn, st_h = _h_sum(hpart, den.reshape(_NBH, 1, _BH))
    h_out = _bn_apply(h, hn[:N], st_h, gh.reshape(1, D), bh.reshape(1, D),
                      N, 1000)
    return h_out, e_out


def kernel(edge_index, h, e, Wh, We, Wa, att, gamma_h, beta_h, gamma_e,
           beta_e):
    ei2 = edge_index.astype(jnp.int32).reshape(2, NROWS, CH)
    for i in range(L):
        h, e = _layer(ei2, h, e, Wh[i], We[i], Wa[i],
                      att[i], gamma_h[i], beta_h[i], gamma_e[i], beta_e[i])
    return (h, e)


# speedup vs baseline: 6.7334x; 1.0392x over previous
import functools

import jax
import jax.numpy as jnp
from jax import lax
from jax.experimental import pallas as pl
from jax.experimental.pallas import tpu as pltpu
from jax.experimental.pallas import tpu_sc as plsc

N = 10000
E = 320000
D = 128
L = 2
NPAD = 10240

NC = 2
NS = 16
NW = NC * NS
EW = E // NW
CH = 80
NCHW = EW // CH
NROWS = E // CH
_NSL = NPAD // NS

_PREC = lax.Precision.HIGHEST

_mesh = plsc.VectorSubcoreMesh(core_axis_name="c", subcore_axis_name="s")
_SC_PARAMS = pltpu.CompilerParams(use_tc_tiling_on_sc=False)
_SC_PARAMS_NL = pltpu.CompilerParams(use_tc_tiling_on_sc=False,
                                     needs_layout_passes=False)


def _f32(shape):
    return jax.ShapeDtypeStruct(shape, jnp.float32)


def _node_mm_body(h_ref, wa_ref, wh_ref, hwa_ref, hwh2_ref):
    h = h_ref[...]
    hwa_ref[...] = lax.dot_general(h, wa_ref[...], (((1,), (0,)), ((), ())),
                                   precision=_PREC,
                                   preferred_element_type=jnp.float32)
    hwh = lax.dot_general(h, wh_ref[...], (((1,), (0,)), ((), ())),
                          precision=_PREC,
                          preferred_element_type=jnp.float32)
    hwh2_ref[0] = hwh[:, :64]
    hwh2_ref[1] = hwh[:, 64:]


def _node_mm(h, wa, wh):
    bn = 2000
    return pl.pallas_call(
        _node_mm_body,
        grid=(N // bn,),
        in_specs=[
            pl.BlockSpec((bn, D), lambda i: (i, 0)),
            pl.BlockSpec((D, D), lambda i: (0, 0)),
            pl.BlockSpec((D, D), lambda i: (0, 0)),
        ],
        out_specs=[
            pl.BlockSpec((bn, D), lambda i: (i, 0)),
            pl.BlockSpec((NC, bn, 64), lambda i: (0, i, 0)),
        ],
        out_shape=[_f32((N, D)), _f32((NC, N, 64))],
    )(h, wa, wh)


_GA = 2
_NRA = 62


@functools.partial(
    pl.kernel,
    out_type=[_f32((E, D)), _f32((E, D))],
    mesh=_mesh,
    compiler_params=_SC_PARAMS,
    scratch_types=[
        pltpu.VMEM((2, _GA, CH), jnp.int32),
        pltpu.VMEM((2, _GA, CH), jnp.int32),
        pltpu.VMEM((_GA * CH, D), jnp.float32),
        pltpu.VMEM((_GA * CH, D), jnp.float32),
        pltpu.VMEM((_GA * CH, D), jnp.float32),
        pltpu.VMEM((_GA * CH, D), jnp.float32),
        pltpu.SemaphoreType.DMA,
        pltpu.SemaphoreType.DMA,
        pltpu.SemaphoreType.DMA,
        pltpu.SemaphoreType.DMA,
    ],
)
def _sc_gather(hwa_hbm, ei_hbm, gs_hbm, gd_hbm,
               iab0, iab1, ras0, rad0, ras1, rad1,
               sg0, sg1, sw0, sw1):
    wid = lax.axis_index("s") * NC + lax.axis_index("c")
    row0 = wid * NCHW
    nb = _GA * CH

    def _issue(r, iab, ras, rad, sg):
        rr = row0 + r * _GA
        pltpu.sync_copy(ei_hbm.at[:, pl.ds(rr, _GA)], iab)
        for j in range(_GA):
            pltpu.async_copy(hwa_hbm.at[iab.at[0, j]],
                             ras.at[pl.ds(j * CH, CH)], sg)
            pltpu.async_copy(hwa_hbm.at[iab.at[1, j]],
                             rad.at[pl.ds(j * CH, CH)], sg)

    def _drain_g(ras, rad, sg):
        pltpu.make_async_copy(hwa_hbm.at[pl.ds(0, nb)], ras, sg).wait()
        pltpu.make_async_copy(hwa_hbm.at[pl.ds(0, nb)], rad, sg).wait()

    def _write(r, ras, rad, sw):
        b = (row0 + r * _GA) * CH
        pltpu.async_copy(ras, gs_hbm.at[pl.ds(b, nb)], sw)
        pltpu.async_copy(rad, gd_hbm.at[pl.ds(b, nb)], sw)

    def _drain_w(ras, rad, sw):
        pltpu.make_async_copy(ras, gs_hbm.at[pl.ds(0, nb)], sw).wait()
        pltpu.make_async_copy(rad, gd_hbm.at[pl.ds(0, nb)], sw).wait()

    _issue(0, iab0, ras0, rad0, sg0)
    _issue(1, iab1, ras1, rad1, sg1)

    @pl.loop(0, _NRA // 2)
    def _(j):
        ra = 2 * j
        rb = 2 * j + 1
        _drain_g(ras0, rad0, sg0)
        _write(ra, ras0, rad0, sw0)
        _drain_g(ras1, rad1, sg1)
        _write(rb, ras1, rad1, sw1)

        @pl.when(ra + 2 < _NRA)
        def _():
            _drain_w(ras0, rad0, sw0)
            _issue(ra + 2, iab0, ras0, rad0, sg0)

        @pl.when(rb + 2 < _NRA)
        def _():
            _drain_w(ras1, rad1, sw1)
            _issue(rb + 2, iab1, ras1, rad1, sg1)

    _drain_w(ras0, rad0, sw0)
    _drain_w(ras1, rad1, sw1)

    rt = row0 + _NRA * _GA
    pltpu.sync_copy(ei_hbm.at[:, pl.ds(rt, 1)], iab0.at[:, pl.ds(0, 1)])
    pltpu.async_copy(hwa_hbm.at[iab0.at[0, 0]], ras0.at[pl.ds(0, CH)], sg0)
    pltpu.async_copy(hwa_hbm.at[iab0.at[1, 0]], rad0.at[pl.ds(0, CH)], sg0)
    pltpu.make_async_copy(hwa_hbm.at[pl.ds(0, CH)],
                          ras0.at[pl.ds(0, CH)], sg0).wait()
    pltpu.make_async_copy(hwa_hbm.at[pl.ds(0, CH)],
                          rad0.at[pl.ds(0, CH)], sg0).wait()
    pltpu.sync_copy(ras0.at[pl.ds(0, CH)], gs_hbm.at[pl.ds(rt * CH, CH)])
    pltpu.sync_copy(rad0.at[pl.ds(0, CH)], gd_hbm.at[pl.ds(rt * CH, CH)])


_BE = 1600
_NBE = E // _BE


def _edge_dense_body(e_ref, gs_ref, gd_ref, we_ref, att_ref,
                     en_ref, s_ref, st_ref, acc_ref):
    i = pl.program_id(0)
    en = lax.dot_general(e_ref[...], we_ref[...], (((1,), (0,)), ((), ())),
                         precision=lax.Precision.DEFAULT,
                         preferred_element_type=jnp.float32)
    en = en + gs_ref[...].astype(jnp.float32) + gd_ref[...].astype(jnp.float32)
    en_ref[...] = en.astype(jnp.bfloat16)
    lr = jnp.where(en > 0, en, 0.2 * en)
    s = lax.dot_general(lr, att_ref[...], (((1,), (1,)), ((), ())),
                        precision=lax.Precision.DEFAULT,
                        preferred_element_type=jnp.float32)
    s_ref[0, 0, :] = s[:, 0]

    @pl.when(i == 0)
    def _():
        acc_ref[...] = jnp.zeros_like(acc_ref)

    acc_ref[0, :] += jnp.sum(en, axis=0)
    acc_ref[1, :] += jnp.sum(en * en, axis=0)

    @pl.when(i == _NBE - 1)
    def _():
        st_ref[...] = acc_ref[...]


def _edge_dense(e, gs, gd, we, att):
    return pl.pallas_call(
        _edge_dense_body,
        grid=(_NBE,),
        in_specs=[
            pl.BlockSpec((_BE, D), lambda i: (i, 0)),
            pl.BlockSpec((_BE, D), lambda i: (i, 0)),
            pl.BlockSpec((_BE, D), lambda i: (i, 0)),
            pl.BlockSpec((D, D), lambda i: (0, 0)),
            pl.BlockSpec((1, D), lambda i: (0, 0)),
        ],
        out_specs=[
            pl.BlockSpec((_BE, D), lambda i: (i, 0)),
            pl.BlockSpec((1, 1, _BE), lambda i: (i, 0, 0)),
            pl.BlockSpec((8, D), lambda i: (0, 0)),
        ],
        out_shape=[jax.ShapeDtypeStruct((E, D), jnp.bfloat16),
                   _f32((_NBE, 1, _BE)), _f32((8, D))],
        scratch_shapes=[pltpu.VMEM((8, D), jnp.float32)],
    )(e, gs, gd, we, att)


_RT = NROWS // NS
_GC = 5
_NR2 = _RT // _GC
_DH = D // NC


@functools.partial(
    pl.kernel,
    out_type=[_f32((NC, NPAD, _DH)), _f32((NPAD,))],
    mesh=_mesh,
    compiler_params=_SC_PARAMS_NL,
    scratch_types=[
        pltpu.VMEM((2, _GC, CH), jnp.int32),
        pltpu.VMEM((_GC, CH), jnp.float32),
        pltpu.VMEM((2, _GC, CH), jnp.int32),
        pltpu.VMEM((_GC, CH), jnp.float32),
        pltpu.VMEM((2, _GC, CH), jnp.int32),
        pltpu.VMEM((_GC, CH), jnp.float32),
        pltpu.VMEM((_GC * CH, _DH), jnp.float32),
        pltpu.VMEM((_GC * CH, _DH), jnp.float32),
        pltpu.VMEM((_GC * CH, _DH), jnp.float32),
        pltpu.VMEM((_NSL,), jnp.float32),
        pltpu.VMEM_SHARED((NPAD,), jnp.float32),
        pltpu.VMEM_SHARED((NPAD, _DH), jnp.float32),
        pltpu.SemaphoreType.DMA,
        pltpu.SemaphoreType.DMA,
        pltpu.SemaphoreType.DMA,
        pltpu.SemaphoreType.DMA,
        pltpu.SemaphoreType.DMA,
        pltpu.SemaphoreType.DMA,
    ],
)
def _sc_message(hwh_hbm, s_hbm, ei_hbm, hpart_hbm, den_hbm,
                bid0, be0, bid1, be1, bid2, be2, r0, r1, r2, zb,
                den_sh, hacc_sh, sg0, sg1, sg2, ss0, ss1, ss2):
    c = lax.axis_index("c")
    sid = lax.axis_index("s")
    row0 = sid * _RT
    coff = c * N

    @pl.loop(0, _NSL, step=16)
    def _(j):
        zb[pl.ds(j, 16)] = jnp.zeros((16,), jnp.float32)

    pltpu.sync_copy(zb, den_sh.at[pl.ds(sid * _NSL, _NSL)])

    @pl.loop(0, _GC * CH)
    def _(j):
        for k in range(_DH // 16):
            r0[j, pl.ds(k * 16, 16)] = jnp.zeros((16,), jnp.float32)

    pltpu.sync_copy(r0.at[pl.ds(0, 400)],
                    hacc_sh.at[pl.ds(sid * _NSL, 400)])
    pltpu.sync_copy(r0.at[pl.ds(0, 240)],
                    hacc_sh.at[pl.ds(sid * _NSL + 400, 240)])

    plsc.subcore_barrier()

    def _drain_scat(bid, be, buf, ss):
        for j in range(_GC):
            pltpu.make_async_copy(buf.at[pl.ds(j * CH, CH)],
                                  hacc_sh.at[bid.at[1, j]], ss).wait()
            pltpu.make_async_copy(be.at[j], den_sh.at[bid.at[1, j]], ss).wait()

    def _issue(r, bid, be, buf, sg, ss, first):
        if not first:
            _drain_scat(bid, be, buf, ss)
        rr = row0 + r * _GC
        pltpu.sync_copy(ei_hbm.at[:, pl.ds(rr, _GC)], bid)
        pltpu.sync_copy(s_hbm.at[pl.ds(rr, _GC)], be)

        @pl.loop(0, _GC)
        def _(j):
            for k in range(CH // 16):
                sl = pl.ds(k * 16, 16)
                bid[0, j, sl] = bid[0, j, sl] + coff
                be[j, sl] = jnp.exp(be[j, sl])

        for j in range(_GC):
            pltpu.async_copy(hwh_hbm.at[bid.at[0, j]],
                             buf.at[pl.ds(j * CH, CH)], sg)

    def _drain_g(buf, sg):
        pltpu.make_async_copy(hwh_hbm.at[pl.ds(0, _GC * CH)], buf, sg).wait()

    def _scale_scatter(bid, be, buf, ss):
        for j in range(_GC):
            @pl.loop(0, CH, unroll=4)
            def _(t):
                a16 = plsc.load_gather(be.at[j],
                                       [jnp.full((16,), t, jnp.int32)])
                for k in range(_DH // 16):
                    sl = pl.ds(k * 16, 16)
                    buf[j * CH + t, sl] = buf[j * CH + t, sl] * a16

        for j in range(_GC):
            pltpu.async_copy(buf.at[pl.ds(j * CH, CH)],
                             hacc_sh.at[bid.at[1, j]], ss, add=True)
            pltpu.async_copy(be.at[j], den_sh.at[bid.at[1, j]], ss, add=True)

    _issue(0, bid0, be0, r0, sg0, ss0, True)
    _issue(1, bid1, be1, r1, sg1, ss1, True)
    _issue(2, bid2, be2, r2, sg2, ss2, True)

    @pl.loop(0, 16)
    def _(i):
        r = 3 * i
        _drain_g(r0, sg0)
        _scale_scatter(bid0, be0, r0, ss0)
        _drain_g(r1, sg1)
        _scale_scatter(bid1, be1, r1, ss1)
        _issue(r + 3, bid0, be0, r0, sg0, ss0, False)
        _drain_g(r2, sg2)
        _scale_scatter(bid2, be2, r2, ss2)
        _issue(r + 4, bid1, be1, r1, sg1, ss1, False)

        @pl.when(i < 15)
        def _():
            _issue(r + 5, bid2, be2, r2, sg2, ss2, False)

    _drain_g(r0, sg0)
    _scale_scatter(bid0, be0, r0, ss0)
    _drain_g(r1, sg1)
    _scale_scatter(bid1, be1, r1, ss1)

    _drain_scat(bid0, be0, r0, ss0)
    _drain_scat(bid1, be1, r1, ss1)
    _drain_scat(bid2, be2, r2, ss2)
    plsc.subcore_barrier()

    @pl.when(c == 0)
    def _():
        pltpu.sync_copy(den_sh.at[pl.ds(sid * _NSL, _NSL)], zb)
        pltpu.sync_copy(zb, den_hbm.at[pl.ds(sid * _NSL, _NSL)])

    pltpu.sync_copy(hacc_sh.at[pl.ds(sid * _NSL, 400)],
                    r0.at[pl.ds(0, 400)])
    pltpu.sync_copy(r0.at[pl.ds(0, 400)],
                    hpart_hbm.at[c, pl.ds(sid * _NSL, 400)])
    pltpu.sync_copy(hacc_sh.at[pl.ds(sid * _NSL + 400, 240)],
                    r0.at[pl.ds(0, 240)])
    pltpu.sync_copy(r0.at[pl.ds(0, 240)],
                    hpart_hbm.at[c, pl.ds(sid * _NSL + 400, 240)])


def _bn_apply_body(n_rows, x_ref, xn_ref, st_ref, g_ref, b_ref, out_ref):
    mu = st_ref[0, :] / n_rows
    var = st_ref[1, :] / n_rows - mu * mu
    inv = lax.rsqrt(var + 1e-5)
    scale = g_ref[0, :] * inv
    shift = b_ref[0, :] - mu * scale
    y = xn_ref[...].astype(jnp.float32) * scale[None, :] + shift[None, :]
    out_ref[...] = x_ref[...] + jnp.maximum(y, 0.0)


def _bn_apply(x, xn, st, g, b, n_rows, bm):
    m = x.shape[0]
    return pl.pallas_call(
        functools.partial(_bn_apply_body, float(n_rows)),
        grid=(m // bm,),
        in_specs=[
            pl.BlockSpec((bm, D), lambda i: (i, 0)),
            pl.BlockSpec((bm, D), lambda i: (i, 0)),
            pl.BlockSpec((8, D), lambda i: (0, 0)),
            pl.BlockSpec((1, D), lambda i: (0, 0)),
            pl.BlockSpec((1, D), lambda i: (0, 0)),
        ],
        out_specs=pl.BlockSpec((bm, D), lambda i: (i, 0)),
        out_shape=_f32((m, D)),
    )(x, xn, st, g, b)


_BH = 1280
_NBH = NPAD // _BH


def _h_sum_body(hp_ref, den_ref, hn_ref, st_ref, acc_ref):
    i = pl.program_id(0)
    hn = jnp.concatenate([hp_ref[0], hp_ref[1]], axis=1)
    den = den_ref[0, 0, :]
    rden = jnp.where(den > 0, 1.0 / den, 0.0)
    hn = hn * rden[:, None]
    hn_ref[...] = hn

    @pl.when(i == 0)
    def _():
        acc_ref[...] = jnp.zeros_like(acc_ref)

    acc_ref[0, :] += jnp.sum(hn, axis=0)
    acc_ref[1, :] += jnp.sum(hn * hn, axis=0)

    @pl.when(i == _NBH - 1)
    def _():
        st_ref[...] = acc_ref[...]


def _h_sum(hpart, den):
    return pl.pallas_call(
        _h_sum_body,
        grid=(_NBH,),
        in_specs=[
            pl.BlockSpec((NC, _BH, _DH), lambda i: (0, i, 0)),
            pl.BlockSpec((1, 1, _BH), lambda i: (i, 0, 0)),
        ],
        out_specs=[
            pl.BlockSpec((_BH, D), lambda i: (i, 0)),
            pl.BlockSpec((8, D), lambda i: (0, 0)),
        ],
        out_shape=[_f32((NPAD, D)), _f32((8, D))],
        scratch_shapes=[pltpu.VMEM((8, D), jnp.float32)],
    )(hpart, den)


def _layer(ei2, h, e, wh, we, wa, att, gh, bh, ge, be):
    hwa, hwh = _node_mm(h, wa, wh)
    gs, gd = _sc_gather(hwa, ei2)
    en, s3, st_e = _edge_dense(e, gs, gd, we, att.reshape(1, D))
    s2 = s3.reshape(NROWS, CH)
    hpart, den = _sc_message(hwh.reshape(NC * N, _DH), s2, ei2)
    e_out = _bn_apply(e, en, st_e, ge.reshape(1, D), be.reshape(1, D), E, _BE)
    hn, st_h = _h_sum(hpart, den.reshape(_NBH, 1, _BH))
    h_out = _bn_apply(h, hn[:N], st_h, gh.reshape(1, D), bh.reshape(1, D),
                      N, 1000)
    return h_out, e_out


def kernel(edge_index, h, e, Wh, We, Wa, att, gamma_h, beta_h, gamma_e,
           beta_e):
    ei2 = edge_index.astype(jnp.int32).reshape(2, NROWS, CH)
    for i in range(L):
        h, e = _layer(ei2, h, e, Wh[i], We[i], Wa[i],
                      att[i], gamma_h[i], beta_h[i], gamma_e[i], beta_e[i])
    return (h, e)


# speedup vs baseline: 6.7363x; 1.0004x over previous
import functools

import jax
import jax.numpy as jnp
from jax import lax
from jax.experimental import pallas as pl
from jax.experimental.pallas import tpu as pltpu
from jax.experimental.pallas import tpu_sc as plsc

N = 10000
E = 320000
D = 128
L = 2
NPAD = 10240

NC = 2
NS = 16
NW = NC * NS
EW = E // NW
CH = 80
NCHW = EW // CH
NROWS = E // CH
_NSL = NPAD // NS

_PREC = lax.Precision.HIGHEST

_mesh = plsc.VectorSubcoreMesh(core_axis_name="c", subcore_axis_name="s")
_SC_PARAMS = pltpu.CompilerParams(use_tc_tiling_on_sc=False)
_SC_PARAMS_NL = pltpu.CompilerParams(use_tc_tiling_on_sc=False,
                                     needs_layout_passes=False)


def _f32(shape):
    return jax.ShapeDtypeStruct(shape, jnp.float32)


def _node_mm_body(h_ref, wa_ref, wh_ref, hwa_ref, hwh2_ref):
    h = h_ref[...]
    hwa_ref[...] = lax.dot_general(h, wa_ref[...], (((1,), (0,)), ((), ())),
                                   precision=_PREC,
                                   preferred_element_type=jnp.float32)
    hwh = lax.dot_general(h, wh_ref[...], (((1,), (0,)), ((), ())),
                          precision=_PREC,
                          preferred_element_type=jnp.float32)
    hwh2_ref[0] = hwh[:, :64]
    hwh2_ref[1] = hwh[:, 64:]


def _node_mm(h, wa, wh):
    bn = 2000
    return pl.pallas_call(
        _node_mm_body,
        grid=(N // bn,),
        in_specs=[
            pl.BlockSpec((bn, D), lambda i: (i, 0)),
            pl.BlockSpec((D, D), lambda i: (0, 0)),
            pl.BlockSpec((D, D), lambda i: (0, 0)),
        ],
        out_specs=[
            pl.BlockSpec((bn, D), lambda i: (i, 0)),
            pl.BlockSpec((NC, bn, 64), lambda i: (0, i, 0)),
        ],
        out_shape=[_f32((N, D)), _f32((NC, N, 64))],
    )(h, wa, wh)


_GA = 2
_NRA = 62


@functools.partial(
    pl.kernel,
    out_type=[_f32((E, D)), _f32((E, D))],
    mesh=_mesh,
    compiler_params=_SC_PARAMS,
    scratch_types=[
        pltpu.VMEM((2, _GA, CH), jnp.int32),
        pltpu.VMEM((2, _GA, CH), jnp.int32),
        pltpu.VMEM((2, _GA, CH), jnp.int32),
        pltpu.VMEM((_GA * CH, D), jnp.float32),
        pltpu.VMEM((_GA * CH, D), jnp.float32),
        pltpu.VMEM((_GA * CH, D), jnp.float32),
        pltpu.VMEM((_GA * CH, D), jnp.float32),
        pltpu.VMEM((_GA * CH, D), jnp.float32),
        pltpu.VMEM((_GA * CH, D), jnp.float32),
        pltpu.SemaphoreType.DMA,
        pltpu.SemaphoreType.DMA,
        pltpu.SemaphoreType.DMA,
        pltpu.SemaphoreType.DMA,
        pltpu.SemaphoreType.DMA,
        pltpu.SemaphoreType.DMA,
    ],
)
def _sc_gather(hwa_hbm, ei_hbm, gs_hbm, gd_hbm,
               iab0, iab1, iab2, ras0, rad0, ras1, rad1, ras2, rad2,
               sg0, sg1, sg2, sw0, sw1, sw2):
    wid = lax.axis_index("s") * NC + lax.axis_index("c")
    row0 = wid * NCHW
    nb = _GA * CH

    def _issue(r, iab, ras, rad, sg):
        rr = row0 + r * _GA
        pltpu.sync_copy(ei_hbm.at[:, pl.ds(rr, _GA)], iab)
        for j in range(_GA):
            pltpu.async_copy(hwa_hbm.at[iab.at[0, j]],
                             ras.at[pl.ds(j * CH, CH)], sg)
            pltpu.async_copy(hwa_hbm.at[iab.at[1, j]],
                             rad.at[pl.ds(j * CH, CH)], sg)

    def _drain_g(ras, rad, sg):
        pltpu.make_async_copy(hwa_hbm.at[pl.ds(0, nb)], ras, sg).wait()
        pltpu.make_async_copy(hwa_hbm.at[pl.ds(0, nb)], rad, sg).wait()

    def _write(r, ras, rad, sw):
        b = (row0 + r * _GA) * CH
        pltpu.async_copy(ras, gs_hbm.at[pl.ds(b, nb)], sw)
        pltpu.async_copy(rad, gd_hbm.at[pl.ds(b, nb)], sw)

    def _drain_w(ras, rad, sw):
        pltpu.make_async_copy(ras, gs_hbm.at[pl.ds(0, nb)], sw).wait()
        pltpu.make_async_copy(rad, gd_hbm.at[pl.ds(0, nb)], sw).wait()

    _issue(0, iab0, ras0, rad0, sg0)
    _issue(1, iab1, ras1, rad1, sg1)
    _issue(2, iab2, ras2, rad2, sg2)

    @pl.loop(0, 20)
    def _(i):
        r = 3 * i
        _drain_g(ras0, rad0, sg0)
        _write(r, ras0, rad0, sw0)
        _drain_g(ras1, rad1, sg1)
        _write(r + 1, ras1, rad1, sw1)
        _drain_w(ras0, rad0, sw0)
        _issue(r + 3, iab0, ras0, rad0, sg0)
        _drain_g(ras2, rad2, sg2)
        _write(r + 2, ras2, rad2, sw2)
        _drain_w(ras1, rad1, sw1)
        _issue(r + 4, iab1, ras1, rad1, sg1)

        @pl.when(i < 19)
        def _():
            _drain_w(ras2, rad2, sw2)
            _issue(r + 5, iab2, ras2, rad2, sg2)

    _drain_g(ras0, rad0, sg0)
    _write(60, ras0, rad0, sw0)
    _drain_g(ras1, rad1, sg1)
    _write(61, ras1, rad1, sw1)
    _drain_w(ras0, rad0, sw0)
    _drain_w(ras1, rad1, sw1)
    _drain_w(ras2, rad2, sw2)

    rt = row0 + _NRA * _GA
    pltpu.sync_copy(ei_hbm.at[:, pl.ds(rt, 1)], iab0.at[:, pl.ds(0, 1)])
    pltpu.async_copy(hwa_hbm.at[iab0.at[0, 0]], ras0.at[pl.ds(0, CH)], sg0)
    pltpu.async_copy(hwa_hbm.at[iab0.at[1, 0]], rad0.at[pl.ds(0, CH)], sg0)
    pltpu.make_async_copy(hwa_hbm.at[pl.ds(0, CH)],
                          ras0.at[pl.ds(0, CH)], sg0).wait()
    pltpu.make_async_copy(hwa_hbm.at[pl.ds(0, CH)],
                          rad0.at[pl.ds(0, CH)], sg0).wait()
    pltpu.sync_copy(ras0.at[pl.ds(0, CH)], gs_hbm.at[pl.ds(rt * CH, CH)])
    pltpu.sync_copy(rad0.at[pl.ds(0, CH)], gd_hbm.at[pl.ds(rt * CH, CH)])


_BE = 1600
_NBE = E // _BE


def _edge_dense_body(e_ref, gs_ref, gd_ref, we_ref, att_ref,
                     en_ref, s_ref, st_ref, acc_ref):
    i = pl.program_id(0)
    en = lax.dot_general(e_ref[...], we_ref[...], (((1,), (0,)), ((), ())),
                         precision=lax.Precision.DEFAULT,
                         preferred_element_type=jnp.float32)
    en = en + gs_ref[...].astype(jnp.float32) + gd_ref[...].astype(jnp.float32)
    en_ref[...] = en.astype(jnp.bfloat16)
    lr = jnp.where(en > 0, en, 0.2 * en)
    s = lax.dot_general(lr, att_ref[...], (((1,), (1,)), ((), ())),
                        precision=lax.Precision.DEFAULT,
                        preferred_element_type=jnp.float32)
    s_ref[0, 0, :] = s[:, 0]

    @pl.when(i == 0)
    def _():
        acc_ref[...] = jnp.zeros_like(acc_ref)

    acc_ref[0, :] += jnp.sum(en, axis=0)
    acc_ref[1, :] += jnp.sum(en * en, axis=0)

    @pl.when(i == _NBE - 1)
    def _():
        st_ref[...] = acc_ref[...]


def _edge_dense(e, gs, gd, we, att):
    return pl.pallas_call(
        _edge_dense_body,
        grid=(_NBE,),
        in_specs=[
            pl.BlockSpec((_BE, D), lambda i: (i, 0)),
            pl.BlockSpec((_BE, D), lambda i: (i, 0)),
            pl.BlockSpec((_BE, D), lambda i: (i, 0)),
            pl.BlockSpec((D, D), lambda i: (0, 0)),
            pl.BlockSpec((1, D), lambda i: (0, 0)),
        ],
        out_specs=[
            pl.BlockSpec((_BE, D), lambda i: (i, 0)),
            pl.BlockSpec((1, 1, _BE), lambda i: (i, 0, 0)),
            pl.BlockSpec((8, D), lambda i: (0, 0)),
        ],
        out_shape=[jax.ShapeDtypeStruct((E, D), jnp.bfloat16),
                   _f32((_NBE, 1, _BE)), _f32((8, D))],
        scratch_shapes=[pltpu.VMEM((8, D), jnp.float32)],
    )(e, gs, gd, we, att)


_RT = NROWS // NS
_GC = 5
_NR2 = _RT // _GC
_DH = D // NC


@functools.partial(
    pl.kernel,
    out_type=[_f32((NC, NPAD, _DH)), _f32((NPAD,))],
    mesh=_mesh,
    compiler_params=_SC_PARAMS_NL,
    scratch_types=[
        pltpu.VMEM((2, _GC, CH), jnp.int32),
        pltpu.VMEM((_GC, CH), jnp.float32),
        pltpu.VMEM((2, _GC, CH), jnp.int32),
        pltpu.VMEM((_GC, CH), jnp.float32),
        pltpu.VMEM((2, _GC, CH), jnp.int32),
        pltpu.VMEM((_GC, CH), jnp.float32),
        pltpu.VMEM((_GC * CH, _DH), jnp.float32),
        pltpu.VMEM((_GC * CH, _DH), jnp.float32),
        pltpu.VMEM((_GC * CH, _DH), jnp.float32),
        pltpu.VMEM((_NSL,), jnp.float32),
        pltpu.VMEM_SHARED((NPAD,), jnp.float32),
        pltpu.VMEM_SHARED((NPAD, _DH), jnp.float32),
        pltpu.SemaphoreType.DMA,
        pltpu.SemaphoreType.DMA,
        pltpu.SemaphoreType.DMA,
        pltpu.SemaphoreType.DMA,
        pltpu.SemaphoreType.DMA,
        pltpu.SemaphoreType.DMA,
    ],
)
def _sc_message(hwh_hbm, s_hbm, ei_hbm, hpart_hbm, den_hbm,
                bid0, be0, bid1, be1, bid2, be2, r0, r1, r2, zb,
                den_sh, hacc_sh, sg0, sg1, sg2, ss0, ss1, ss2):
    c = lax.axis_index("c")
    sid = lax.axis_index("s")
    row0 = sid * _RT
    coff = c * N

    @pl.loop(0, _NSL, step=16)
    def _(j):
        zb[pl.ds(j, 16)] = jnp.zeros((16,), jnp.float32)

    pltpu.sync_copy(zb, den_sh.at[pl.ds(sid * _NSL, _NSL)])

    @pl.loop(0, _GC * CH)
    def _(j):
        for k in range(_DH // 16):
            r0[j, pl.ds(k * 16, 16)] = jnp.zeros((16,), jnp.float32)

    pltpu.sync_copy(r0.at[pl.ds(0, 400)],
                    hacc_sh.at[pl.ds(sid * _NSL, 400)])
    pltpu.sync_copy(r0.at[pl.ds(0, 240)],
                    hacc_sh.at[pl.ds(sid * _NSL + 400, 240)])

    plsc.subcore_barrier()

    def _drain_scat(bid, be, buf, ss):
        for j in range(_GC):
            pltpu.make_async_copy(buf.at[pl.ds(j * CH, CH)],
                                  hacc_sh.at[bid.at[1, j]], ss).wait()
            pltpu.make_async_copy(be.at[j], den_sh.at[bid.at[1, j]], ss).wait()

    def _issue(r, bid, be, buf, sg, ss, first):
        if not first:
            _drain_scat(bid, be, buf, ss)
        rr = row0 + r * _GC
        pltpu.sync_copy(ei_hbm.at[:, pl.ds(rr, _GC)], bid)
        pltpu.sync_copy(s_hbm.at[pl.ds(rr, _GC)], be)

        @pl.loop(0, _GC)
        def _(j):
            for k in range(CH // 16):
                sl = pl.ds(k * 16, 16)
                bid[0, j, sl] = bid[0, j, sl] + coff
                be[j, sl] = jnp.exp(be[j, sl])

        for j in range(_GC):
            pltpu.async_copy(hwh_hbm.at[bid.at[0, j]],
                             buf.at[pl.ds(j * CH, CH)], sg)

    def _drain_g(buf, sg):
        pltpu.make_async_copy(hwh_hbm.at[pl.ds(0, _GC * CH)], buf, sg).wait()

    def _scale_scatter(bid, be, buf, ss):
        for j in range(_GC):
            @pl.loop(0, CH, unroll=4)
            def _(t):
                a16 = plsc.load_gather(be.at[j],
                                       [jnp.full((16,), t, jnp.int32)])
                for k in range(_DH // 16):
                    sl = pl.ds(k * 16, 16)
                    buf[j * CH + t, sl] = buf[j * CH + t, sl] * a16

        for j in range(_GC):
            pltpu.async_copy(buf.at[pl.ds(j * CH, CH)],
                             hacc_sh.at[bid.at[1, j]], ss, add=True)
            pltpu.async_copy(be.at[j], den_sh.at[bid.at[1, j]], ss, add=True)

    _issue(0, bid0, be0, r0, sg0, ss0, True)
    _issue(1, bid1, be1, r1, sg1, ss1, True)
    _issue(2, bid2, be2, r2, sg2, ss2, True)

    @pl.loop(0, 16)
    def _(i):
        r = 3 * i
        _drain_g(r0, sg0)
        _scale_scatter(bid0, be0, r0, ss0)
        _drain_g(r1, sg1)
        _scale_scatter(bid1, be1, r1, ss1)
        _issue(r + 3, bid0, be0, r0, sg0, ss0, False)
        _drain_g(r2, sg2)
        _scale_scatter(bid2, be2, r2, ss2)
        _issue(r + 4, bid1, be1, r1, sg1, ss1, False)

        @pl.when(i < 15)
        def _():
            _issue(r + 5, bid2, be2, r2, sg2, ss2, False)

    _drain_g(r0, sg0)
    _scale_scatter(bid0, be0, r0, ss0)
    _drain_g(r1, sg1)
    _scale_scatter(bid1, be1, r1, ss1)

    _drain_scat(bid0, be0, r0, ss0)
    _drain_scat(bid1, be1, r1, ss1)
    _drain_scat(bid2, be2, r2, ss2)
    plsc.subcore_barrier()

    @pl.when(c == 0)
    def _():
        pltpu.sync_copy(den_sh.at[pl.ds(sid * _NSL, _NSL)], zb)
        pltpu.sync_copy(zb, den_hbm.at[pl.ds(sid * _NSL, _NSL)])

    pltpu.sync_copy(hacc_sh.at[pl.ds(sid * _NSL, 400)],
                    r0.at[pl.ds(0, 400)])
    pltpu.sync_copy(r0.at[pl.ds(0, 400)],
                    hpart_hbm.at[c, pl.ds(sid * _NSL, 400)])
    pltpu.sync_copy(hacc_sh.at[pl.ds(sid * _NSL + 400, 240)],
                    r0.at[pl.ds(0, 240)])
    pltpu.sync_copy(r0.at[pl.ds(0, 240)],
                    hpart_hbm.at[c, pl.ds(sid * _NSL + 400, 240)])


def _bn_apply_body(n_rows, x_ref, xn_ref, st_ref, g_ref, b_ref, out_ref):
    mu = st_ref[0, :] / n_rows
    var = st_ref[1, :] / n_rows - mu * mu
    inv = lax.rsqrt(var + 1e-5)
    scale = g_ref[0, :] * inv
    shift = b_ref[0, :] - mu * scale
    y = xn_ref[...].astype(jnp.float32) * scale[None, :] + shift[None, :]
    out_ref[...] = x_ref[...] + jnp.maximum(y, 0.0)


def _bn_apply(x, xn, st, g, b, n_rows, bm):
    m = x.shape[0]
    return pl.pallas_call(
        functools.partial(_bn_apply_body, float(n_rows)),
        grid=(m // bm,),
        in_specs=[
            pl.BlockSpec((bm, D), lambda i: (i, 0)),
            pl.BlockSpec((bm, D), lambda i: (i, 0)),
            pl.BlockSpec((8, D), lambda i: (0, 0)),
            pl.BlockSpec((1, D), lambda i: (0, 0)),
            pl.BlockSpec((1, D), lambda i: (0, 0)),
        ],
        out_specs=pl.BlockSpec((bm, D), lambda i: (i, 0)),
        out_shape=_f32((m, D)),
    )(x, xn, st, g, b)


_BH = 1280
_NBH = NPAD // _BH


def _h_sum_body(hp_ref, den_ref, hn_ref, st_ref, acc_ref):
    i = pl.program_id(0)
    hn = jnp.concatenate([hp_ref[0], hp_ref[1]], axis=1)
    den = den_ref[0, 0, :]
    rden = jnp.where(den > 0, 1.0 / den, 0.0)
    hn = hn * rden[:, None]
    hn_ref[...] = hn

    @pl.when(i == 0)
    def _():
        acc_ref[...] = jnp.zeros_like(acc_ref)

    acc_ref[0, :] += jnp.sum(hn, axis=0)
    acc_ref[1, :] += jnp.sum(hn * hn, axis=0)

    @pl.when(i == _NBH - 1)
    def _():
        st_ref[...] = acc_ref[...]


def _h_sum(hpart, den):
    return pl.pallas_call(
        _h_sum_body,
        grid=(_NBH,),
        in_specs=[
            pl.BlockSpec((NC, _BH, _DH), lambda i: (0, i, 0)),
            pl.BlockSpec((1, 1, _BH), lambda i: (i, 0, 0)),
        ],
        out_specs=[
            pl.BlockSpec((_BH, D), lambda i: (i, 0)),
            pl.BlockSpec((8, D), lambda i: (0, 0)),
        ],
        out_shape=[_f32((NPAD, D)), _f32((8, D))],
        scratch_shapes=[pltpu.VMEM((8, D), jnp.float32)],
    )(hpart, den)


def _layer(ei2, h, e, wh, we, wa, att, gh, bh, ge, be):
    hwa, hwh = _node_mm(h, wa, wh)
    gs, gd = _sc_gather(hwa, ei2)
    en, s3, st_e = _edge_dense(e, gs, gd, we, att.reshape(1, D))
    s2 = s3.reshape(NROWS, CH)
    hpart, den = _sc_message(hwh.reshape(NC * N, _DH), s2, ei2)
    e_out = _bn_apply(e, en, st_e, ge.reshape(1, D), be.reshape(1, D), E, _BE)
    hn, st_h = _h_sum(hpart, den.reshape(_NBH, 1, _BH))
    h_out = _bn_apply(h, hn[:N], st_h, gh.reshape(1, D), bh.reshape(1, D),
                      N, 1000)
    return h_out, e_out


def kernel(edge_index, h, e, Wh, We, Wa, att, gamma_h, beta_h, gamma_e,
           beta_e):
    ei2 = edge_index.astype(jnp.int32).reshape(2, NROWS, CH)
    for i in range(L):
        h, e = _layer(ei2, h, e, Wh[i], We[i], Wa[i],
                      att[i], gamma_h[i], beta_h[i], gamma_e[i], beta_e[i])
    return (h, e)
